# Initial kernel scaffold; baseline (speedup 1.0000x reference)
import jax
import jax.numpy as jnp
from jax.experimental import pallas as pl


def kernel(x, edge_index, batch, W_l1, b_l1, W_r1, W_l2, b_l2, W_r2, W_ih, W_hh, b_ih, b_hh):
    raise NotImplementedError("write your pallas kernel here")


# speedup vs baseline: 3.3266x; 3.3266x over previous
import functools

import jax
import jax.numpy as jnp
from jax import lax
from jax.experimental import pallas as pl
from jax.experimental.pallas import tpu as pltpu
from jax.experimental.pallas import tpu_sc as plsc

T = 16
N = 10000
E = 160000
D = 256
O = 256
NCORES = 2
NSUB = 16
EPT = E // NSUB
CH = 80
NCHUNK = EPT // CH
ROW_CH = 640
ROW_TAIL = N - ROW_CH * (NSUB - 1)
DEG_CH = 640
DEG_TAIL = N - DEG_CH * (NSUB - 1)
M = T * N


def _sc_segsum_body(with_deg, *refs):
    if with_deg:
        (table, srcs, dsts, zeros2d, zeros1d, ones1d, s_out, deg_out,
         src_v, dst_v, rows_v, ones_v, deg_v, dego_v, acc, deg_acc, sem) = refs
    else:
        (table, srcs, dsts, zeros2d, s_out,
         src_v, dst_v, rows_v, ones_v, deg_v, dego_v, acc, deg_acc, sem) = refs
    c = lax.axis_index("c")
    s = lax.axis_index("s")
    last = s == NSUB - 1
    row_off = pl.multiple_of(s * ROW_CH, 8)
    deg_off = pl.multiple_of(s * DEG_CH, 8)

    if with_deg:
        pltpu.sync_copy(ones1d, ones_v)
        pltpu.sync_copy(zeros1d, deg_v)

    @pl.loop(0, T)
    def _t(t):
        @pl.when(jnp.logical_not(last))
        def _():
            pltpu.sync_copy(zeros2d, acc.at[pl.ds(row_off, ROW_CH)])
            if with_deg:
                pltpu.sync_copy(deg_v, deg_acc.at[pl.ds(deg_off, DEG_CH)])

        @pl.when(last)
        def _():
            pltpu.sync_copy(zeros2d.at[pl.ds(0, ROW_TAIL)],
                            acc.at[pl.ds(ROW_CH * (NSUB - 1), ROW_TAIL)])
            if with_deg:
                pltpu.sync_copy(deg_v.at[pl.ds(0, DEG_TAIL)],
                                deg_acc.at[pl.ds(DEG_CH * (NSUB - 1), DEG_TAIL)])
        plsc.subcore_barrier()

        pltpu.sync_copy(srcs.at[c, t, s], src_v)
        pltpu.sync_copy(dsts.at[t, s], dst_v)

        @pl.loop(0, NCHUNK)
        def _chunk(i):
            pltpu.async_copy(table.at[src_v.at[i]], rows_v, sem).wait()
            pltpu.sync_copy(rows_v, acc.at[dst_v.at[i]], add=True)
            if with_deg:
                pltpu.sync_copy(ones_v, deg_acc.at[dst_v.at[i]], add=True)
        plsc.subcore_barrier()

        @pl.when(jnp.logical_not(last))
        def _():
            pltpu.sync_copy(acc.at[pl.ds(row_off, ROW_CH)],
                            s_out.at[c, t, pl.ds(row_off, ROW_CH)])
            if with_deg:
                @pl.when(c == 0)
                def _():
                    pltpu.sync_copy(deg_acc.at[pl.ds(deg_off, DEG_CH)], dego_v)
                    pltpu.sync_copy(
                        dego_v,
                        deg_out.at[pl.ds(pl.multiple_of(t * N + deg_off, 8),
                                         DEG_CH)])

        @pl.when(last)
        def _():
            pltpu.sync_copy(acc.at[pl.ds(ROW_CH * (NSUB - 1), ROW_TAIL)],
                            s_out.at[c, t, pl.ds(ROW_CH * (NSUB - 1), ROW_TAIL)])
            if with_deg:
                @pl.when(c == 0)
                def _():
                    pltpu.sync_copy(deg_acc.at[pl.ds(DEG_CH * (NSUB - 1),
                                                     DEG_TAIL)],
                                    dego_v.at[pl.ds(0, DEG_TAIL)])
                    pltpu.sync_copy(dego_v.at[pl.ds(0, DEG_TAIL)],
                                    deg_out.at[pl.ds(pl.multiple_of(
                                        t * N + DEG_CH * (NSUB - 1), 8),
                                        DEG_TAIL)])


def _sc_segsum(table, srcs, dsts, with_deg):
    zeros2d = jnp.zeros((ROW_CH, 128), jnp.float32)
    zeros1d = jnp.zeros((DEG_CH,), jnp.float32)
    ones1d = jnp.ones((CH,), jnp.float32)
    out_type = [jax.ShapeDtypeStruct((NCORES, T, N, 128), jnp.float32)]
    if with_deg:
        out_type.append(jax.ShapeDtypeStruct((T * N,), jnp.float32))
    mesh = plsc.VectorSubcoreMesh(core_axis_name="c", subcore_axis_name="s",
                                  num_cores=NCORES, num_subcores=NSUB)
    scratch = [
        pltpu.VMEM((NCHUNK, CH), jnp.int32),
        pltpu.VMEM((NCHUNK, CH), jnp.int32),
        pltpu.VMEM((CH, 128), jnp.float32),
        pltpu.VMEM((CH,), jnp.float32),
        pltpu.VMEM((DEG_CH,), jnp.float32),
        pltpu.VMEM((DEG_CH,), jnp.float32),
        pltpu.VMEM_SHARED((N, 128), jnp.float32),
        pltpu.VMEM_SHARED((N,), jnp.float32),
        pltpu.SemaphoreType.DMA,
    ]
    kern = pl.kernel(
        functools.partial(_sc_segsum_body, with_deg),
        out_type=tuple(out_type) if with_deg else out_type[0],
        mesh=mesh,
        scratch_types=scratch,
    )
    if with_deg:
        return kern(table, srcs, dsts, zeros2d, zeros1d, ones1d)
    return kern(table, srcs, dsts, zeros2d)


BM = 1280


def _mm1_body(x_ref, wl_ref, wr_ref, bl_ref, a_ref, b_ref):
    xb = x_ref[...]
    a = jnp.dot(xb, wl_ref[...], preferred_element_type=jnp.float32)
    b = jnp.dot(xb, wr_ref[...], preferred_element_type=jnp.float32) + bl_ref[...]
    a_ref[0] = a[:, :128]
    a_ref[1] = a[:, 128:]
    b_ref[0] = b[:, :128]
    b_ref[1] = b[:, 128:]


def _mm1(x, wl, wr, bl):
    grid = (M // BM,)
    return pl.pallas_call(
        _mm1_body,
        grid=grid,
        in_specs=[
            pl.BlockSpec((BM, D), lambda i: (i, 0)),
            pl.BlockSpec((D, O), lambda i: (0, 0)),
            pl.BlockSpec((D, O), lambda i: (0, 0)),
            pl.BlockSpec((1, O), lambda i: (0, 0)),
        ],
        out_specs=[
            pl.BlockSpec((2, BM, 128), lambda i: (0, i, 0)),
            pl.BlockSpec((2, BM, 128), lambda i: (0, i, 0)),
        ],
        out_shape=[
            jax.ShapeDtypeStruct((2, M, 128), jnp.float32),
            jax.ShapeDtypeStruct((2, M, 128), jnp.float32),
        ],
    )(x, wl, wr, bl)


def _mm2_body(s_ref, b1_ref, deg_ref, wl_ref, wr_ref, bl_ref, a_ref, b_ref):
    inv = 1.0 / jnp.maximum(deg_ref[...], 1.0)
    h_lo = jnp.maximum(s_ref[0] * inv + b1_ref[0], 0.0)
    h_hi = jnp.maximum(s_ref[1] * inv + b1_ref[1], 0.0)
    a = (jnp.dot(h_lo, wl_ref[0], preferred_element_type=jnp.float32)
         + jnp.dot(h_hi, wl_ref[1], preferred_element_type=jnp.float32))
    b = (jnp.dot(h_lo, wr_ref[0], preferred_element_type=jnp.float32)
         + jnp.dot(h_hi, wr_ref[1], preferred_element_type=jnp.float32)
         + bl_ref[...])
    a_ref[0] = a[:, :128]
    a_ref[1] = a[:, 128:]
    b_ref[0] = b[:, :128]
    b_ref[1] = b[:, 128:]


def _mm2(s1, b1, deg, wl2s, wr2s, bl2):
    grid = (M // BM,)
    return pl.pallas_call(
        _mm2_body,
        grid=grid,
        in_specs=[
            pl.BlockSpec((2, BM, 128), lambda i: (0, i, 0)),
            pl.BlockSpec((2, BM, 128), lambda i: (0, i, 0)),
            pl.BlockSpec((BM, 1), lambda i: (i, 0)),
            pl.BlockSpec((2, 128, O), lambda i: (0, 0, 0)),
            pl.BlockSpec((2, 128, O), lambda i: (0, 0, 0)),
            pl.BlockSpec((1, O), lambda i: (0, 0)),
        ],
        out_specs=[
            pl.BlockSpec((2, BM, 128), lambda i: (0, i, 0)),
            pl.BlockSpec((2, BM, 128), lambda i: (0, i, 0)),
        ],
        out_shape=[
            jax.ShapeDtypeStruct((2, M, 128), jnp.float32),
            jax.ShapeDtypeStruct((2, M, 128), jnp.float32),
        ],
    )(s1, b1, deg, wl2s, wr2s, bl2)


BN = 2000


def _pool_body(s_ref, b_ref, deg_ref, o_ref):
    j = pl.program_id(1)
    inv = 1.0 / jnp.maximum(deg_ref[0], 1.0)
    v0 = jnp.max(s_ref[0, 0] * inv + b_ref[0, 0], axis=0, keepdims=True)
    v1 = jnp.max(s_ref[1, 0] * inv + b_ref[1, 0], axis=0, keepdims=True)
    val = jnp.concatenate([v0, v1], axis=0)

    @pl.when(j == 0)
    def _():
        o_ref[0] = val

    @pl.when(j > 0)
    def _():
        o_ref[0] = jnp.maximum(o_ref[0], val)


def _pool(s2, b2, deg):
    grid = (T, N // BN)
    return pl.pallas_call(
        _pool_body,
        grid=grid,
        in_specs=[
            pl.BlockSpec((2, 1, BN, 128), lambda t, j: (0, t, j, 0)),
            pl.BlockSpec((2, 1, BN, 128), lambda t, j: (0, t, j, 0)),
            pl.BlockSpec((1, BN, 1), lambda t, j: (t, j, 0)),
        ],
        out_specs=pl.BlockSpec((1, 2, 128), lambda t, j: (t, 0, 0)),
        out_shape=jax.ShapeDtypeStruct((T, 2, 128), jnp.float32),
    )(s2, b2, deg)


def _head_body(z_ref, wih_ref, whh_ref, bih_ref, bhh_ref,
               out_ref, nce_ref, acc_ref):
    z = z_ref[...]
    h = jnp.zeros((1, O), jnp.float32)
    outs = []
    for t in range(T):
        zt = z[t:t + 1, :]
        gi = jnp.dot(zt, wih_ref[...], preferred_element_type=jnp.float32) + bih_ref[...]
        gh = jnp.dot(h, whh_ref[...], preferred_element_type=jnp.float32) + bhh_ref[...]
        r = jax.nn.sigmoid(gi[:, :O] + gh[:, :O])
        zz = jax.nn.sigmoid(gi[:, O:2 * O] + gh[:, O:2 * O])
        n = jnp.tanh(gi[:, 2 * O:] + r * gh[:, 2 * O:])
        h = (1.0 - zz) * n + zz * h
        outs.append(h)
    out_ref[...] = jnp.concatenate(outs, axis=0)

    nce = jnp.float32(0.0)
    correct = jnp.float32(0.0)
    for ts in range(2, 10):
        c_t = outs[ts]
        cn = jnp.maximum(jnp.sqrt(jnp.sum(c_t * c_t)), 1e-8)
        for i in (1, 2):
            tot = []
            for ridx in (ts + i, ts + i + 2, ts + i + 3, ts + i + 4):
                a = z[ridx:ridx + 1, :]
                an = jnp.maximum(jnp.sqrt(jnp.sum(a * a)), 1e-8)
                tot.append(jnp.sum(a * c_t) / (an * cn))
            m = jnp.maximum(jnp.maximum(tot[0], tot[1]),
                            jnp.maximum(tot[2], tot[3]))
            lse = m + jnp.log(jnp.exp(tot[0] - m) + jnp.exp(tot[1] - m)
                              + jnp.exp(tot[2] - m) + jnp.exp(tot[3] - m))
            nce = nce + (tot[0] - lse)
            others = jnp.maximum(tot[1], jnp.maximum(tot[2], tot[3]))
            correct = correct + jnp.where(tot[0] >= others, 1.0, 0.0)
    nce_ref[...] = jnp.full((1, 1), nce / jnp.float32(-16.0), jnp.float32)
    acc_ref[...] = jnp.full((1, 1), correct / jnp.float32(16.0), jnp.float32)


def _head(z, wihT, whhT, bih, bhh):
    return pl.pallas_call(
        _head_body,
        in_specs=[
            pl.BlockSpec((T, O), lambda: (0, 0)),
            pl.BlockSpec((O, 3 * O), lambda: (0, 0)),
            pl.BlockSpec((O, 3 * O), lambda: (0, 0)),
            pl.BlockSpec((1, 3 * O), lambda: (0, 0)),
            pl.BlockSpec((1, 3 * O), lambda: (0, 0)),
        ],
        out_specs=[
            pl.BlockSpec((T, O), lambda: (0, 0)),
            pl.BlockSpec((1, 1), lambda: (0, 0)),
            pl.BlockSpec((1, 1), lambda: (0, 0)),
        ],
        out_shape=[
            jax.ShapeDtypeStruct((T, O), jnp.float32),
            jax.ShapeDtypeStruct((1, 1), jnp.float32),
            jax.ShapeDtypeStruct((1, 1), jnp.float32),
        ],
    )(z, wihT, whhT, bih, bhh)


def kernel(x, edge_index, batch, W_l1, b_l1, W_r1, W_l2, b_l2, W_r2,
           W_ih, W_hh, b_ih, b_hh):
    del batch
    xf = x.reshape(M, D)

    src = (edge_index[:, 0, :]
           + (jnp.arange(T, dtype=jnp.int32) * N)[:, None])
    srcs = jnp.stack([src, src + M]).reshape(2, T, NSUB, NCHUNK, CH)
    dsts = edge_index[:, 1, :].reshape(T, NSUB, NCHUNK, CH)

    a1, b1 = _mm1(xf, W_l1, W_r1, b_l1.reshape(1, O))

    s1, deg = _sc_segsum(a1.reshape(2 * M, 128), srcs, dsts, with_deg=True)
    s1 = s1.reshape(2, M, 128)
    degf = deg.reshape(M, 1)

    wl2s = W_l2.reshape(2, 128, O)
    wr2s = W_r2.reshape(2, 128, O)
    a2, b2 = _mm2(s1, b1, degf, wl2s, wr2s, b_l2.reshape(1, O))

    s2 = _sc_segsum(a2.reshape(2 * M, 128), srcs, dsts, with_deg=False)

    pooled = _pool(s2, b2.reshape(2, T, N, 128), deg.reshape(T, N, 1))
    z = pooled.reshape(T, O)

    gru_out, nce, acc = _head(z, W_ih.T, W_hh.T,
                              b_ih.reshape(1, 3 * O), b_hh.reshape(1, 3 * O))
    return nce[0, 0], acc[0, 0], gru_out[None]


# speedup vs baseline: 5.6564x; 1.7004x over previous
import functools

import jax
import jax.numpy as jnp
from jax import lax
from jax.experimental import pallas as pl
from jax.experimental.pallas import tpu as pltpu
from jax.experimental.pallas import tpu_sc as plsc

T = 16
N = 10000
E = 160000
D = 256
O = 256
NCORES = 2
NSUB = 16
EPT = E // NSUB
CH = 50
NCHUNK = EPT // CH
NGROUP = 40
ROW_CH = 640
ROW_TAIL = N - ROW_CH * (NSUB - 1)
DEG_CH = 640
DEG_TAIL = N - DEG_CH * (NSUB - 1)
NBUF = 5
M = T * N


def _sc_segsum_body(with_deg, *refs):
    if with_deg:
        (table, srcs, dsts, zeros2d, zeros1d, ones1d, s_out, deg_out,
         srcr, dstr, rows_v, ones_v, deg_v, dego_v, acc, deg_acc,
         *sems) = refs
    else:
        (table, srcs, dsts, zeros2d, s_out,
         srcr, dstr, rows_v, ones_v, deg_v, dego_v, acc, deg_acc,
         *sems) = refs
    gsem = sems[:NBUF]
    ssem = sems[NBUF:2 * NBUF]
    dsem = sems[2 * NBUF:3 * NBUF]
    isem0, isem1 = sems[3 * NBUF:3 * NBUF + 2]
    c = lax.axis_index("c")
    s = lax.axis_index("s")
    last = s == NSUB - 1
    row_off = pl.multiple_of(s * ROW_CH, 8)
    deg_off = pl.multiple_of(s * DEG_CH, 8)

    if with_deg:
        pltpu.sync_copy(ones1d, ones_v)
        pltpu.sync_copy(zeros1d, deg_v)

    @pl.loop(0, T)
    def _t(t):
        @pl.when(jnp.logical_not(last))
        def _():
            pltpu.sync_copy(zeros2d, acc.at[pl.ds(row_off, ROW_CH)])
            if with_deg:
                pltpu.sync_copy(deg_v, deg_acc.at[pl.ds(deg_off, DEG_CH)])

        @pl.when(last)
        def _():
            pltpu.sync_copy(zeros2d.at[pl.ds(0, ROW_TAIL)],
                            acc.at[pl.ds(ROW_CH * (NSUB - 1), ROW_TAIL)])
            if with_deg:
                pltpu.sync_copy(deg_v.at[pl.ds(0, DEG_TAIL)],
                                deg_acc.at[pl.ds(DEG_CH * (NSUB - 1), DEG_TAIL)])
        plsc.subcore_barrier()

        pltpu.sync_copy(srcs.at[c, t, s, 0], srcr.at[0])
        pltpu.sync_copy(dsts.at[t, s, 0], dstr.at[0])
        pltpu.async_copy(srcs.at[c, t, s, 1], srcr.at[1], isem0)
        pltpu.async_copy(dsts.at[t, s, 1], dstr.at[1], isem1)
        for b in range(NBUF):
            pltpu.async_copy(table.at[srcr.at[0, b]], rows_v.at[b], gsem[b])

        @pl.loop(0, NGROUP - 1)
        def _grp(k):
            p = lax.rem(k, 2)
            pn = 1 - p
            pltpu.make_async_copy(srcs.at[c, t, s, 0], srcr.at[pn],
                                  isem0).wait()
            pltpu.make_async_copy(dsts.at[t, s, 0], dstr.at[pn],
                                  isem1).wait()
            for b in range(NBUF):
                pltpu.make_async_copy(table.at[srcr.at[p, b]], rows_v.at[b],
                                      gsem[b]).wait()
                sd = pltpu.async_copy(rows_v.at[b], acc.at[dstr.at[p, b]],
                                      ssem[b], add=True)
                if with_deg:
                    dd = pltpu.async_copy(ones_v, deg_acc.at[dstr.at[p, b]],
                                          dsem[b], add=True)
                sd.wait()
                if with_deg:
                    dd.wait()
                pltpu.async_copy(table.at[srcr.at[pn, b]], rows_v.at[b],
                                 gsem[b])
            @pl.when(k + 2 < NGROUP)
            def _():
                pltpu.async_copy(srcs.at[c, t, s, k + 2], srcr.at[p], isem0)
                pltpu.async_copy(dsts.at[t, s, k + 2], dstr.at[p], isem1)

        pf = (NGROUP - 1) % 2
        for b in range(NBUF):
            pltpu.make_async_copy(table.at[srcr.at[pf, b]], rows_v.at[b],
                                  gsem[b]).wait()
            pltpu.sync_copy(rows_v.at[b], acc.at[dstr.at[pf, b]], add=True)
            if with_deg:
                pltpu.sync_copy(ones_v, deg_acc.at[dstr.at[pf, b]], add=True)
        plsc.subcore_barrier()

        @pl.when(jnp.logical_not(last))
        def _():
            pltpu.sync_copy(acc.at[pl.ds(row_off, ROW_CH)],
                            s_out.at[c, t, pl.ds(row_off, ROW_CH)])
            if with_deg:
                @pl.when(c == 0)
                def _():
                    pltpu.sync_copy(deg_acc.at[pl.ds(deg_off, DEG_CH)], dego_v)
                    pltpu.sync_copy(
                        dego_v,
                        deg_out.at[pl.ds(pl.multiple_of(t * N + deg_off, 8),
                                         DEG_CH)])

        @pl.when(last)
        def _():
            pltpu.sync_copy(acc.at[pl.ds(ROW_CH * (NSUB - 1), ROW_TAIL)],
                            s_out.at[c, t, pl.ds(ROW_CH * (NSUB - 1), ROW_TAIL)])
            if with_deg:
                @pl.when(c == 0)
                def _():
                    pltpu.sync_copy(deg_acc.at[pl.ds(DEG_CH * (NSUB - 1),
                                                     DEG_TAIL)],
                                    dego_v.at[pl.ds(0, DEG_TAIL)])
                    pltpu.sync_copy(dego_v.at[pl.ds(0, DEG_TAIL)],
                                    deg_out.at[pl.ds(pl.multiple_of(
                                        t * N + DEG_CH * (NSUB - 1), 8),
                                        DEG_TAIL)])


def _sc_segsum(table, srcs, dsts, with_deg):
    zeros2d = jnp.zeros((ROW_CH, 128), jnp.float32)
    zeros1d = jnp.zeros((DEG_CH,), jnp.float32)
    ones1d = jnp.ones((CH,), jnp.float32)
    out_type = [jax.ShapeDtypeStruct((NCORES, T, N, 128), jnp.float32)]
    if with_deg:
        out_type.append(jax.ShapeDtypeStruct((T * N,), jnp.float32))
    mesh = plsc.VectorSubcoreMesh(core_axis_name="c", subcore_axis_name="s",
                                  num_cores=NCORES, num_subcores=NSUB)
    scratch = [
        pltpu.VMEM((2, NBUF, CH), jnp.int32),
        pltpu.VMEM((2, NBUF, CH), jnp.int32),
        pltpu.VMEM((NBUF, CH, 128), jnp.float32),
        pltpu.VMEM((CH,), jnp.float32),
        pltpu.VMEM((DEG_CH,), jnp.float32),
        pltpu.VMEM((DEG_CH,), jnp.float32),
        pltpu.VMEM_SHARED((N, 128), jnp.float32),
        pltpu.VMEM_SHARED((N,), jnp.float32),
    ] + [pltpu.SemaphoreType.DMA] * (3 * NBUF + 2)
    kern = pl.kernel(
        functools.partial(_sc_segsum_body, with_deg),
        out_type=tuple(out_type) if with_deg else out_type[0],
        mesh=mesh,
        scratch_types=scratch,
    )
    if with_deg:
        return kern(table, srcs, dsts, zeros2d, zeros1d, ones1d)
    return kern(table, srcs, dsts, zeros2d)


BM = 1280


def _mm1_body(x_ref, wl_ref, wr_ref, bl_ref, a_ref, b_ref):
    xb = x_ref[...]
    a = jnp.dot(xb, wl_ref[...], preferred_element_type=jnp.float32)
    b = jnp.dot(xb, wr_ref[...], preferred_element_type=jnp.float32) + bl_ref[...]
    a_ref[0] = a[:, :128]
    a_ref[1] = a[:, 128:]
    b_ref[0] = b[:, :128]
    b_ref[1] = b[:, 128:]


def _mm1(x, wl, wr, bl):
    grid = (M // BM,)
    return pl.pallas_call(
        _mm1_body,
        grid=grid,
        in_specs=[
            pl.BlockSpec((BM, D), lambda i: (i, 0)),
            pl.BlockSpec((D, O), lambda i: (0, 0)),
            pl.BlockSpec((D, O), lambda i: (0, 0)),
            pl.BlockSpec((1, O), lambda i: (0, 0)),
        ],
        out_specs=[
            pl.BlockSpec((2, BM, 128), lambda i: (0, i, 0)),
            pl.BlockSpec((2, BM, 128), lambda i: (0, i, 0)),
        ],
        out_shape=[
            jax.ShapeDtypeStruct((2, M, 128), jnp.float32),
            jax.ShapeDtypeStruct((2, M, 128), jnp.float32),
        ],
    )(x, wl, wr, bl)


def _mm2_body(s_ref, b1_ref, deg_ref, wl_ref, wr_ref, bl_ref, a_ref, b_ref):
    inv = 1.0 / jnp.maximum(deg_ref[...], 1.0)
    h_lo = jnp.maximum(s_ref[0] * inv + b1_ref[0], 0.0)
    h_hi = jnp.maximum(s_ref[1] * inv + b1_ref[1], 0.0)
    a = (jnp.dot(h_lo, wl_ref[0], preferred_element_type=jnp.float32)
         + jnp.dot(h_hi, wl_ref[1], preferred_element_type=jnp.float32))
    b = (jnp.dot(h_lo, wr_ref[0], preferred_element_type=jnp.float32)
         + jnp.dot(h_hi, wr_ref[1], preferred_element_type=jnp.float32)
         + bl_ref[...])
    a_ref[0] = a[:, :128]
    a_ref[1] = a[:, 128:]
    b_ref[0] = b[:, :128]
    b_ref[1] = b[:, 128:]


def _mm2(s1, b1, deg, wl2s, wr2s, bl2):
    grid = (M // BM,)
    return pl.pallas_call(
        _mm2_body,
        grid=grid,
        in_specs=[
            pl.BlockSpec((2, BM, 128), lambda i: (0, i, 0)),
            pl.BlockSpec((2, BM, 128), lambda i: (0, i, 0)),
            pl.BlockSpec((BM, 1), lambda i: (i, 0)),
            pl.BlockSpec((2, 128, O), lambda i: (0, 0, 0)),
            pl.BlockSpec((2, 128, O), lambda i: (0, 0, 0)),
            pl.BlockSpec((1, O), lambda i: (0, 0)),
        ],
        out_specs=[
            pl.BlockSpec((2, BM, 128), lambda i: (0, i, 0)),
            pl.BlockSpec((2, BM, 128), lambda i: (0, i, 0)),
        ],
        out_shape=[
            jax.ShapeDtypeStruct((2, M, 128), jnp.float32),
            jax.ShapeDtypeStruct((2, M, 128), jnp.float32),
        ],
    )(s1, b1, deg, wl2s, wr2s, bl2)


BN = 2000


def _pool_body(s_ref, b_ref, deg_ref, o_ref):
    j = pl.program_id(1)
    inv = 1.0 / jnp.maximum(deg_ref[0], 1.0)
    v0 = jnp.max(s_ref[0, 0] * inv + b_ref[0, 0], axis=0, keepdims=True)
    v1 = jnp.max(s_ref[1, 0] * inv + b_ref[1, 0], axis=0, keepdims=True)
    val = jnp.concatenate([v0, v1], axis=0)

    @pl.when(j == 0)
    def _():
        o_ref[0] = val

    @pl.when(j > 0)
    def _():
        o_ref[0] = jnp.maximum(o_ref[0], val)


def _pool(s2, b2, deg):
    grid = (T, N // BN)
    return pl.pallas_call(
        _pool_body,
        grid=grid,
        in_specs=[
            pl.BlockSpec((2, 1, BN, 128), lambda t, j: (0, t, j, 0)),
            pl.BlockSpec((2, 1, BN, 128), lambda t, j: (0, t, j, 0)),
            pl.BlockSpec((1, BN, 1), lambda t, j: (t, j, 0)),
        ],
        out_specs=pl.BlockSpec((1, 2, 128), lambda t, j: (t, 0, 0)),
        out_shape=jax.ShapeDtypeStruct((T, 2, 128), jnp.float32),
    )(s2, b2, deg)


def _head_body(z_ref, wih_ref, whh_ref, bih_ref, bhh_ref,
               out_ref, nce_ref, acc_ref):
    z = z_ref[...]
    h = jnp.zeros((1, O), jnp.float32)
    outs = []
    for t in range(T):
        zt = z[t:t + 1, :]
        gi = jnp.dot(zt, wih_ref[...], preferred_element_type=jnp.float32) + bih_ref[...]
        gh = jnp.dot(h, whh_ref[...], preferred_element_type=jnp.float32) + bhh_ref[...]
        r = jax.nn.sigmoid(gi[:, :O] + gh[:, :O])
        zz = jax.nn.sigmoid(gi[:, O:2 * O] + gh[:, O:2 * O])
        n = jnp.tanh(gi[:, 2 * O:] + r * gh[:, 2 * O:])
        h = (1.0 - zz) * n + zz * h
        outs.append(h)
    out_ref[...] = jnp.concatenate(outs, axis=0)

    nce = jnp.float32(0.0)
    correct = jnp.float32(0.0)
    for ts in range(2, 10):
        c_t = outs[ts]
        cn = jnp.maximum(jnp.sqrt(jnp.sum(c_t * c_t)), 1e-8)
        for i in (1, 2):
            tot = []
            for ridx in (ts + i, ts + i + 2, ts + i + 3, ts + i + 4):
                a = z[ridx:ridx + 1, :]
                an = jnp.maximum(jnp.sqrt(jnp.sum(a * a)), 1e-8)
                tot.append(jnp.sum(a * c_t) / (an * cn))
            m = jnp.maximum(jnp.maximum(tot[0], tot[1]),
                            jnp.maximum(tot[2], tot[3]))
            lse = m + jnp.log(jnp.exp(tot[0] - m) + jnp.exp(tot[1] - m)
                              + jnp.exp(tot[2] - m) + jnp.exp(tot[3] - m))
            nce = nce + (tot[0] - lse)
            others = jnp.maximum(tot[1], jnp.maximum(tot[2], tot[3]))
            correct = correct + jnp.where(tot[0] >= others, 1.0, 0.0)
    nce_ref[...] = jnp.full((1, 1), nce / jnp.float32(-16.0), jnp.float32)
    acc_ref[...] = jnp.full((1, 1), correct / jnp.float32(16.0), jnp.float32)


def _head(z, wihT, whhT, bih, bhh):
    return pl.pallas_call(
        _head_body,
        in_specs=[
            pl.BlockSpec((T, O), lambda: (0, 0)),
            pl.BlockSpec((O, 3 * O), lambda: (0, 0)),
            pl.BlockSpec((O, 3 * O), lambda: (0, 0)),
            pl.BlockSpec((1, 3 * O), lambda: (0, 0)),
            pl.BlockSpec((1, 3 * O), lambda: (0, 0)),
        ],
        out_specs=[
            pl.BlockSpec((T, O), lambda: (0, 0)),
            pl.BlockSpec((1, 1), lambda: (0, 0)),
            pl.BlockSpec((1, 1), lambda: (0, 0)),
        ],
        out_shape=[
            jax.ShapeDtypeStruct((T, O), jnp.float32),
            jax.ShapeDtypeStruct((1, 1), jnp.float32),
            jax.ShapeDtypeStruct((1, 1), jnp.float32),
        ],
    )(z, wihT, whhT, bih, bhh)


def kernel(x, edge_index, batch, W_l1, b_l1, W_r1, W_l2, b_l2, W_r2,
           W_ih, W_hh, b_ih, b_hh):
    del batch
    xf = x.reshape(M, D)

    src = (edge_index[:, 0, :]
           + (jnp.arange(T, dtype=jnp.int32) * N)[:, None])
    srcs = jnp.stack([src, src + M]).reshape(2, T, NSUB, NGROUP, NBUF, CH)
    dsts = edge_index[:, 1, :].reshape(T, NSUB, NGROUP, NBUF, CH)

    a1, b1 = _mm1(xf, W_l1, W_r1, b_l1.reshape(1, O))

    s1, deg = _sc_segsum(a1.reshape(2 * M, 128), srcs, dsts, with_deg=True)
    s1 = s1.reshape(2, M, 128)
    degf = deg.reshape(M, 1)

    wl2s = W_l2.reshape(2, 128, O)
    wr2s = W_r2.reshape(2, 128, O)
    a2, b2 = _mm2(s1, b1, degf, wl2s, wr2s, b_l2.reshape(1, O))

    s2 = _sc_segsum(a2.reshape(2 * M, 128), srcs, dsts, with_deg=False)

    pooled = _pool(s2, b2.reshape(2, T, N, 128), deg.reshape(T, N, 1))
    z = pooled.reshape(T, O)

    gru_out, nce, acc = _head(z, W_ih.T, W_hh.T,
                              b_ih.reshape(1, 3 * O), b_hh.reshape(1, 3 * O))
    return nce[0, 0], acc[0, 0], gru_out[None]


# speedup vs baseline: 5.9932x; 1.0595x over previous
import functools

import jax
import jax.numpy as jnp
from jax import lax
from jax.experimental import pallas as pl
from jax.experimental.pallas import tpu as pltpu
from jax.experimental.pallas import tpu_sc as plsc

T = 16
N = 10000
E = 160000
D = 256
O = 256
NCORES = 2
NSUB = 16
EPT = E // NSUB
CH = 50
NCHUNK = EPT // CH
NGROUP = 40
ROW_CH = 640
ROW_TAIL = N - ROW_CH * (NSUB - 1)
DEG_CH = 640
DEG_TAIL = N - DEG_CH * (NSUB - 1)
NBUF = 5
ZB = 40
M = T * N
TSTEPS = T // 2
HM = TSTEPS * N


def _sc_segsum_body(with_deg, t0, *refs):
    if with_deg:
        (table, srcs, dsts, zeros2d, zeros1d, ones1d, s_out, deg_out,
         srcr, dstr, rows_v, ones_v, deg_v, dego_v, zv_v, acc, deg_acc,
         *sems) = refs
    else:
        (table, srcs, dsts, zeros2d, s_out,
         srcr, dstr, rows_v, ones_v, deg_v, dego_v, zv_v, acc, deg_acc,
         *sems) = refs
    gsem = sems[:NBUF]
    ssem = sems[NBUF:2 * NBUF]
    dsem = sems[2 * NBUF:3 * NBUF]
    isem0, isem1 = sems[3 * NBUF:3 * NBUF + 2]
    c = lax.axis_index("c")
    s = lax.axis_index("s")
    last = s == NSUB - 1
    row_off = pl.multiple_of(s * ROW_CH, 8)
    deg_off = pl.multiple_of(s * DEG_CH, 8)

    pltpu.sync_copy(zeros2d, zv_v)
    if with_deg:
        pltpu.sync_copy(ones1d, ones_v)
        pltpu.sync_copy(zeros1d, deg_v)

    @pl.loop(0, TSTEPS)
    def _t(t):
        @pl.when(jnp.logical_not(last))
        def _():
            for i in range(ROW_CH // ZB):
                pltpu.sync_copy(zv_v, acc.at[pl.ds(row_off + i * ZB, ZB)])
            if with_deg:
                pltpu.sync_copy(deg_v, deg_acc.at[pl.ds(deg_off, DEG_CH)])

        @pl.when(last)
        def _():
            for i in range(ROW_TAIL // ZB):
                pltpu.sync_copy(zv_v, acc.at[pl.ds(ROW_CH * (NSUB - 1)
                                                   + i * ZB, ZB)])
            if with_deg:
                pltpu.sync_copy(deg_v.at[pl.ds(0, DEG_TAIL)],
                                deg_acc.at[pl.ds(DEG_CH * (NSUB - 1), DEG_TAIL)])
        plsc.subcore_barrier()

        ta = t0 + t
        pltpu.sync_copy(srcs.at[c, ta, s, 0], srcr.at[0])
        pltpu.sync_copy(dsts.at[ta, s, 0], dstr.at[0])
        pltpu.async_copy(srcs.at[c, ta, s, 1], srcr.at[1], isem0)
        pltpu.async_copy(dsts.at[ta, s, 1], dstr.at[1], isem1)
        for b in range(NBUF):
            pltpu.async_copy(table.at[srcr.at[0, b]], rows_v.at[b], gsem[b])

        @pl.loop(0, NGROUP - 1)
        def _grp(k):
            p = lax.rem(k, 2)
            pn = 1 - p
            pltpu.make_async_copy(srcs.at[c, ta, s, 0], srcr.at[pn],
                                  isem0).wait()
            pltpu.make_async_copy(dsts.at[ta, s, 0], dstr.at[pn],
                                  isem1).wait()
            for b in range(NBUF):
                pltpu.make_async_copy(table.at[srcr.at[p, b]], rows_v.at[b],
                                      gsem[b]).wait()
                sd = pltpu.async_copy(rows_v.at[b], acc.at[dstr.at[p, b]],
                                      ssem[b], add=True)
                if with_deg:
                    dd = pltpu.async_copy(ones_v, deg_acc.at[dstr.at[p, b]],
                                          dsem[b], add=True)
                sd.wait()
                if with_deg:
                    dd.wait()
                pltpu.async_copy(table.at[srcr.at[pn, b]], rows_v.at[b],
                                 gsem[b])
            @pl.when(k + 2 < NGROUP)
            def _():
                pltpu.async_copy(srcs.at[c, ta, s, k + 2], srcr.at[p], isem0)
                pltpu.async_copy(dsts.at[ta, s, k + 2], dstr.at[p], isem1)

        pf = (NGROUP - 1) % 2
        for b in range(NBUF):
            pltpu.make_async_copy(table.at[srcr.at[pf, b]], rows_v.at[b],
                                  gsem[b]).wait()
            pltpu.sync_copy(rows_v.at[b], acc.at[dstr.at[pf, b]], add=True)
            if with_deg:
                pltpu.sync_copy(ones_v, deg_acc.at[dstr.at[pf, b]], add=True)
        plsc.subcore_barrier()

        @pl.when(jnp.logical_not(last))
        def _():
            pltpu.sync_copy(acc.at[pl.ds(row_off, ROW_CH)],
                            s_out.at[c, t, pl.ds(row_off, ROW_CH)])
            if with_deg:
                @pl.when(c == 0)
                def _():
                    pltpu.sync_copy(deg_acc.at[pl.ds(deg_off, DEG_CH)], dego_v)
                    pltpu.sync_copy(
                        dego_v,
                        deg_out.at[pl.ds(pl.multiple_of(t * N + deg_off, 8),
                                         DEG_CH)])

        @pl.when(last)
        def _():
            pltpu.sync_copy(acc.at[pl.ds(ROW_CH * (NSUB - 1), ROW_TAIL)],
                            s_out.at[c, t, pl.ds(ROW_CH * (NSUB - 1), ROW_TAIL)])
            if with_deg:
                @pl.when(c == 0)
                def _():
                    pltpu.sync_copy(deg_acc.at[pl.ds(DEG_CH * (NSUB - 1),
                                                     DEG_TAIL)],
                                    dego_v.at[pl.ds(0, DEG_TAIL)])
                    pltpu.sync_copy(dego_v.at[pl.ds(0, DEG_TAIL)],
                                    deg_out.at[pl.ds(pl.multiple_of(
                                        t * N + DEG_CH * (NSUB - 1), 8),
                                        DEG_TAIL)])


def _sc_segsum(table, srcs, dsts, t0, with_deg):
    zeros2d = jnp.zeros((ZB, 128), jnp.float32)
    zeros1d = jnp.zeros((DEG_CH,), jnp.float32)
    ones1d = jnp.ones((CH,), jnp.float32)
    out_type = [jax.ShapeDtypeStruct((NCORES, TSTEPS, N, 128), jnp.float32)]
    if with_deg:
        out_type.append(jax.ShapeDtypeStruct((TSTEPS * N,), jnp.float32))
    mesh = plsc.VectorSubcoreMesh(core_axis_name="c", subcore_axis_name="s",
                                  num_cores=NCORES, num_subcores=NSUB)
    scratch = [
        pltpu.VMEM((2, NBUF, CH), jnp.int32),
        pltpu.VMEM((2, NBUF, CH), jnp.int32),
        pltpu.VMEM((NBUF, CH, 128), jnp.float32),
        pltpu.VMEM((CH,), jnp.float32),
        pltpu.VMEM((DEG_CH,), jnp.float32),
        pltpu.VMEM((DEG_CH,), jnp.float32),
        pltpu.VMEM((ZB, 128), jnp.float32),
        pltpu.VMEM_SHARED((N, 128), jnp.float32),
        pltpu.VMEM_SHARED((N,), jnp.float32),
    ] + [pltpu.SemaphoreType.DMA] * (3 * NBUF + 2)
    kern = pl.kernel(
        functools.partial(_sc_segsum_body, with_deg, t0),
        out_type=tuple(out_type) if with_deg else out_type[0],
        mesh=mesh,
        scratch_types=scratch,
    )
    if with_deg:
        return kern(table, srcs, dsts, zeros2d, zeros1d, ones1d)
    return kern(table, srcs, dsts, zeros2d)


BM = 1600
HB = HM // BM


def _mm1_body(x_ref, wl_ref, wr_ref, bl_ref, a_ref, b_ref):
    xb = x_ref[...]
    a = jnp.dot(xb, wl_ref[...], preferred_element_type=jnp.float32)
    b = jnp.dot(xb, wr_ref[...], preferred_element_type=jnp.float32) + bl_ref[...]
    a_ref[0] = a[:, :128]
    a_ref[1] = a[:, 128:]
    b_ref[0] = b[:, :128]
    b_ref[1] = b[:, 128:]


def _mm1(x, wl, wr, bl, h):
    return pl.pallas_call(
        _mm1_body,
        grid=(HB,),
        in_specs=[
            pl.BlockSpec((BM, D), lambda i: (i + h * HB, 0)),
            pl.BlockSpec((D, O), lambda i: (0, 0)),
            pl.BlockSpec((D, O), lambda i: (0, 0)),
            pl.BlockSpec((1, O), lambda i: (0, 0)),
        ],
        out_specs=[
            pl.BlockSpec((2, BM, 128), lambda i: (0, i, 0)),
            pl.BlockSpec((2, BM, 128), lambda i: (0, i, 0)),
        ],
        out_shape=[
            jax.ShapeDtypeStruct((2, HM, 128), jnp.float32),
            jax.ShapeDtypeStruct((2, HM, 128), jnp.float32),
        ],
    )(x, wl, wr, bl)


def _mm2_body(s_ref, b1_ref, deg_ref, wl_ref, wr_ref, bl_ref, a_ref, b_ref):
    inv = 1.0 / jnp.maximum(deg_ref[...], 1.0)
    h_lo = jnp.maximum(s_ref[0] * inv + b1_ref[0], 0.0)
    h_hi = jnp.maximum(s_ref[1] * inv + b1_ref[1], 0.0)
    a = (jnp.dot(h_lo, wl_ref[0], preferred_element_type=jnp.float32)
         + jnp.dot(h_hi, wl_ref[1], preferred_element_type=jnp.float32))
    b = (jnp.dot(h_lo, wr_ref[0], preferred_element_type=jnp.float32)
         + jnp.dot(h_hi, wr_ref[1], preferred_element_type=jnp.float32)
         + bl_ref[...])
    a_ref[0] = a[:, :128]
    a_ref[1] = a[:, 128:]
    b_ref[0] = b[:, :128]
    b_ref[1] = b[:, 128:]


def _mm2(s1, b1, deg, wl2s, wr2s, bl2):
    return pl.pallas_call(
        _mm2_body,
        grid=(HB,),
        in_specs=[
            pl.BlockSpec((2, BM, 128), lambda i: (0, i, 0)),
            pl.BlockSpec((2, BM, 128), lambda i: (0, i, 0)),
            pl.BlockSpec((BM, 1), lambda i: (i, 0)),
            pl.BlockSpec((2, 128, O), lambda i: (0, 0, 0)),
            pl.BlockSpec((2, 128, O), lambda i: (0, 0, 0)),
            pl.BlockSpec((1, O), lambda i: (0, 0)),
        ],
        out_specs=[
            pl.BlockSpec((2, BM, 128), lambda i: (0, i, 0)),
            pl.BlockSpec((2, BM, 128), lambda i: (0, i, 0)),
        ],
        out_shape=[
            jax.ShapeDtypeStruct((2, HM, 128), jnp.float32),
            jax.ShapeDtypeStruct((2, HM, 128), jnp.float32),
        ],
    )(s1, b1, deg, wl2s, wr2s, bl2)


BN = 2000


def _pool_body(s_ref, b_ref, deg_ref, o_ref):
    j = pl.program_id(1)
    inv = 1.0 / jnp.maximum(deg_ref[0], 1.0)
    v0 = jnp.max(s_ref[0, 0] * inv + b_ref[0, 0], axis=0, keepdims=True)
    v1 = jnp.max(s_ref[1, 0] * inv + b_ref[1, 0], axis=0, keepdims=True)
    val = jnp.concatenate([v0, v1], axis=0)

    @pl.when(j == 0)
    def _():
        o_ref[0] = val

    @pl.when(j > 0)
    def _():
        o_ref[0] = jnp.maximum(o_ref[0], val)


def _pool(s2, b2, deg):
    grid = (TSTEPS, N // BN)
    return pl.pallas_call(
        _pool_body,
        grid=grid,
        in_specs=[
            pl.BlockSpec((2, 1, BN, 128), lambda t, j: (0, t, j, 0)),
            pl.BlockSpec((2, 1, BN, 128), lambda t, j: (0, t, j, 0)),
            pl.BlockSpec((1, BN, 1), lambda t, j: (t, j, 0)),
        ],
        out_specs=pl.BlockSpec((1, 2, 128), lambda t, j: (t, 0, 0)),
        out_shape=jax.ShapeDtypeStruct((TSTEPS, 2, 128), jnp.float32),
    )(s2, b2, deg)


def _head_body(z_ref, wih_ref, whh_ref, bih_ref, bhh_ref,
               out_ref, nce_ref, acc_ref):
    z = z_ref[...]
    h = jnp.zeros((1, O), jnp.float32)
    outs = []
    for t in range(T):
        zt = z[t:t + 1, :]
        gi = jnp.dot(zt, wih_ref[...], preferred_element_type=jnp.float32) + bih_ref[...]
        gh = jnp.dot(h, whh_ref[...], preferred_element_type=jnp.float32) + bhh_ref[...]
        r = jax.nn.sigmoid(gi[:, :O] + gh[:, :O])
        zz = jax.nn.sigmoid(gi[:, O:2 * O] + gh[:, O:2 * O])
        n = jnp.tanh(gi[:, 2 * O:] + r * gh[:, 2 * O:])
        h = (1.0 - zz) * n + zz * h
        outs.append(h)
    out_ref[...] = jnp.concatenate(outs, axis=0)

    nce = jnp.float32(0.0)
    correct = jnp.float32(0.0)
    for ts in range(2, 10):
        c_t = outs[ts]
        cn = jnp.maximum(jnp.sqrt(jnp.sum(c_t * c_t)), 1e-8)
        for i in (1, 2):
            tot = []
            for ridx in (ts + i, ts + i + 2, ts + i + 3, ts + i + 4):
                a = z[ridx:ridx + 1, :]
                an = jnp.maximum(jnp.sqrt(jnp.sum(a * a)), 1e-8)
                tot.append(jnp.sum(a * c_t) / (an * cn))
            m = jnp.maximum(jnp.maximum(tot[0], tot[1]),
                            jnp.maximum(tot[2], tot[3]))
            lse = m + jnp.log(jnp.exp(tot[0] - m) + jnp.exp(tot[1] - m)
                              + jnp.exp(tot[2] - m) + jnp.exp(tot[3] - m))
            nce = nce + (tot[0] - lse)
            others = jnp.maximum(tot[1], jnp.maximum(tot[2], tot[3]))
            correct = correct + jnp.where(tot[0] >= others, 1.0, 0.0)
    nce_ref[...] = jnp.full((1, 1), nce / jnp.float32(-16.0), jnp.float32)
    acc_ref[...] = jnp.full((1, 1), correct / jnp.float32(16.0), jnp.float32)


def _head(z, wihT, whhT, bih, bhh):
    return pl.pallas_call(
        _head_body,
        in_specs=[
            pl.BlockSpec((T, O), lambda: (0, 0)),
            pl.BlockSpec((O, 3 * O), lambda: (0, 0)),
            pl.BlockSpec((O, 3 * O), lambda: (0, 0)),
            pl.BlockSpec((1, 3 * O), lambda: (0, 0)),
            pl.BlockSpec((1, 3 * O), lambda: (0, 0)),
        ],
        out_specs=[
            pl.BlockSpec((T, O), lambda: (0, 0)),
            pl.BlockSpec((1, 1), lambda: (0, 0)),
            pl.BlockSpec((1, 1), lambda: (0, 0)),
        ],
        out_shape=[
            jax.ShapeDtypeStruct((T, O), jnp.float32),
            jax.ShapeDtypeStruct((1, 1), jnp.float32),
            jax.ShapeDtypeStruct((1, 1), jnp.float32),
        ],
    )(z, wihT, whhT, bih, bhh)


def kernel(x, edge_index, batch, W_l1, b_l1, W_r1, W_l2, b_l2, W_r2,
           W_ih, W_hh, b_ih, b_hh):
    del batch
    xf = x.reshape(M, D)

    src = (edge_index[:, 0, :]
           + ((jnp.arange(T, dtype=jnp.int32) % TSTEPS) * N)[:, None])
    srcs = jnp.stack([src, src + HM]).reshape(2, T, NSUB, NGROUP, NBUF, CH)
    dsts = edge_index[:, 1, :].reshape(T, NSUB, NGROUP, NBUF, CH)

    wl2s = W_l2.reshape(2, 128, O)
    wr2s = W_r2.reshape(2, 128, O)
    bl1 = b_l1.reshape(1, O)
    bl2 = b_l2.reshape(1, O)

    pooled = []
    for h in (0, 1):
        a1, b1 = _mm1(xf, W_l1, W_r1, bl1, h)
        s1, deg = _sc_segsum(a1.reshape(2 * HM, 128), srcs, dsts,
                             h * TSTEPS, with_deg=True)
        a2, b2 = _mm2(s1.reshape(2, HM, 128), b1, deg.reshape(HM, 1),
                      wl2s, wr2s, bl2)
        s2 = _sc_segsum(a2.reshape(2 * HM, 128), srcs, dsts,
                        h * TSTEPS, with_deg=False)
        pooled.append(_pool(s2, b2.reshape(2, TSTEPS, N, 128),
                            deg.reshape(TSTEPS, N, 1)))
    z = jnp.concatenate([p.reshape(TSTEPS, O) for p in pooled], axis=0)

    gru_out, nce, acc = _head(z, W_ih.T, W_hh.T,
                              b_ih.reshape(1, 3 * O), b_hh.reshape(1, 3 * O))
    return nce[0, 0], acc[0, 0], gru_out[None]


# speedup vs baseline: 6.1072x; 1.0190x over previous
import functools

import jax
import jax.numpy as jnp
from jax import lax
from jax.experimental import pallas as pl
from jax.experimental.pallas import tpu as pltpu
from jax.experimental.pallas import tpu_sc as plsc

T = 16
N = 10000
E = 160000
D = 256
O = 256
NCORES = 2
NSUB = 16
EPT = E // NSUB
CH = 50
NCHUNK = EPT // CH
NGROUP = 40
ROW_CH = 640
ROW_TAIL = N - ROW_CH * (NSUB - 1)
DEG_CH = 640
DEG_TAIL = N - DEG_CH * (NSUB - 1)
NBUF = 5
ZB = 40
M = T * N
TSTEPS = T // 2
HM = TSTEPS * N


def _sc_segsum_body(with_deg, t0, *refs):
    if with_deg:
        (table, srcs, dsts, zeros2d, zeros1d, ones1d, s_out, deg_out,
         srcr, dstr, rows_v, ones_v, deg_v, dego_v, zv_v, acc, deg_acc,
         *sems) = refs
    else:
        (table, srcs, dsts, zeros2d, s_out,
         srcr, dstr, rows_v, ones_v, deg_v, dego_v, zv_v, acc, deg_acc,
         *sems) = refs
    gsem = sems[:NBUF]
    ssem = sems[NBUF:2 * NBUF]
    dsem = sems[2 * NBUF:3 * NBUF]
    isem0, isem1, csem = sems[3 * NBUF:3 * NBUF + 3]
    c = lax.axis_index("c")
    s = lax.axis_index("s")
    last = s == NSUB - 1
    row_off = pl.multiple_of(s * ROW_CH, 8)
    deg_off = pl.multiple_of(s * DEG_CH, 8)

    pltpu.sync_copy(zeros2d, zv_v)
    if with_deg:
        pltpu.sync_copy(ones1d, ones_v)
        pltpu.sync_copy(zeros1d, deg_v)

    def _zero():
        @pl.when(jnp.logical_not(last))
        def _():
            for i in range(ROW_CH // ZB):
                pltpu.sync_copy(zv_v, acc.at[pl.ds(row_off + i * ZB, ZB)])
            if with_deg:
                pltpu.sync_copy(deg_v, deg_acc.at[pl.ds(deg_off, DEG_CH)])

        @pl.when(last)
        def _():
            for i in range(ROW_TAIL // ZB):
                pltpu.sync_copy(zv_v, acc.at[pl.ds(ROW_CH * (NSUB - 1)
                                                   + i * ZB, ZB)])
            if with_deg:
                pltpu.sync_copy(deg_v.at[pl.ds(0, DEG_TAIL)],
                                deg_acc.at[pl.ds(DEG_CH * (NSUB - 1), DEG_TAIL)])

    def _prologue(ta):
        pltpu.sync_copy(srcs.at[c, ta, s, 0], srcr.at[0])
        pltpu.sync_copy(dsts.at[ta, s, 0], dstr.at[0])
        pltpu.async_copy(srcs.at[c, ta, s, 1], srcr.at[1], isem0)
        pltpu.async_copy(dsts.at[ta, s, 1], dstr.at[1], isem1)
        for b in range(NBUF):
            pltpu.async_copy(table.at[srcr.at[0, b]], rows_v.at[b], gsem[b])

    _zero()
    plsc.subcore_barrier()
    _prologue(t0)

    @pl.loop(0, TSTEPS)
    def _t(t):
        ta = t0 + t

        @pl.loop(0, NGROUP - 1)
        def _grp(k):
            p = lax.rem(k, 2)
            pn = 1 - p
            pltpu.make_async_copy(srcs.at[c, ta, s, 0], srcr.at[pn],
                                  isem0).wait()
            pltpu.make_async_copy(dsts.at[ta, s, 0], dstr.at[pn],
                                  isem1).wait()
            for b in range(NBUF):
                pltpu.make_async_copy(table.at[srcr.at[p, b]], rows_v.at[b],
                                      gsem[b]).wait()
                sd = pltpu.async_copy(rows_v.at[b], acc.at[dstr.at[p, b]],
                                      ssem[b], add=True)
                if with_deg:
                    dd = pltpu.async_copy(ones_v, deg_acc.at[dstr.at[p, b]],
                                          dsem[b], add=True)
                sd.wait()
                if with_deg:
                    dd.wait()
                pltpu.async_copy(table.at[srcr.at[pn, b]], rows_v.at[b],
                                 gsem[b])
            @pl.when(k + 2 < NGROUP)
            def _():
                pltpu.async_copy(srcs.at[c, ta, s, k + 2], srcr.at[p], isem0)
                pltpu.async_copy(dsts.at[ta, s, k + 2], dstr.at[p], isem1)

        pf = (NGROUP - 1) % 2
        for b in range(NBUF):
            pltpu.make_async_copy(table.at[srcr.at[pf, b]], rows_v.at[b],
                                  gsem[b]).wait()
            pltpu.sync_copy(rows_v.at[b], acc.at[dstr.at[pf, b]], add=True)
            if with_deg:
                pltpu.sync_copy(ones_v, deg_acc.at[dstr.at[pf, b]], add=True)
        plsc.subcore_barrier()

        @pl.when(jnp.logical_not(last))
        def _():
            pltpu.async_copy(acc.at[pl.ds(row_off, ROW_CH)],
                             s_out.at[c, t, pl.ds(row_off, ROW_CH)], csem)
            if with_deg:
                @pl.when(c == 0)
                def _():
                    pltpu.sync_copy(deg_acc.at[pl.ds(deg_off, DEG_CH)], dego_v)
                    pltpu.sync_copy(
                        dego_v,
                        deg_out.at[pl.ds(pl.multiple_of(t * N + deg_off, 8),
                                         DEG_CH)])

        @pl.when(last)
        def _():
            pltpu.async_copy(acc.at[pl.ds(ROW_CH * (NSUB - 1), ROW_TAIL)],
                             s_out.at[c, t, pl.ds(ROW_CH * (NSUB - 1),
                                                  ROW_TAIL)], csem)
            if with_deg:
                @pl.when(c == 0)
                def _():
                    pltpu.sync_copy(deg_acc.at[pl.ds(DEG_CH * (NSUB - 1),
                                                     DEG_TAIL)],
                                    dego_v.at[pl.ds(0, DEG_TAIL)])
                    pltpu.sync_copy(dego_v.at[pl.ds(0, DEG_TAIL)],
                                    deg_out.at[pl.ds(pl.multiple_of(
                                        t * N + DEG_CH * (NSUB - 1), 8),
                                        DEG_TAIL)])

        @pl.when(t + 1 < TSTEPS)
        def _():
            _prologue(ta + 1)

        @pl.when(jnp.logical_not(last))
        def _():
            pltpu.make_async_copy(acc.at[pl.ds(row_off, ROW_CH)],
                                  s_out.at[c, t, pl.ds(row_off, ROW_CH)],
                                  csem).wait()

        @pl.when(last)
        def _():
            pltpu.make_async_copy(acc.at[pl.ds(ROW_CH * (NSUB - 1), ROW_TAIL)],
                                  s_out.at[c, t, pl.ds(ROW_CH * (NSUB - 1),
                                                       ROW_TAIL)], csem).wait()

        @pl.when(t + 1 < TSTEPS)
        def _():
            _zero()
        plsc.subcore_barrier()


def _sc_segsum(table, srcs, dsts, t0, with_deg):
    zeros2d = jnp.zeros((ZB, 128), jnp.float32)
    zeros1d = jnp.zeros((DEG_CH,), jnp.float32)
    ones1d = jnp.ones((CH,), jnp.float32)
    out_type = [jax.ShapeDtypeStruct((NCORES, TSTEPS, N, 128), jnp.float32)]
    if with_deg:
        out_type.append(jax.ShapeDtypeStruct((TSTEPS * N,), jnp.float32))
    mesh = plsc.VectorSubcoreMesh(core_axis_name="c", subcore_axis_name="s",
                                  num_cores=NCORES, num_subcores=NSUB)
    scratch = [
        pltpu.VMEM((2, NBUF, CH), jnp.int32),
        pltpu.VMEM((2, NBUF, CH), jnp.int32),
        pltpu.VMEM((NBUF, CH, 128), jnp.float32),
        pltpu.VMEM((CH,), jnp.float32),
        pltpu.VMEM((DEG_CH,), jnp.float32),
        pltpu.VMEM((DEG_CH,), jnp.float32),
        pltpu.VMEM((ZB, 128), jnp.float32),
        pltpu.VMEM_SHARED((N, 128), jnp.float32),
        pltpu.VMEM_SHARED((N,), jnp.float32),
    ] + [pltpu.SemaphoreType.DMA] * (3 * NBUF + 3)
    kern = pl.kernel(
        functools.partial(_sc_segsum_body, with_deg, t0),
        out_type=tuple(out_type) if with_deg else out_type[0],
        mesh=mesh,
        scratch_types=scratch,
    )
    if with_deg:
        return kern(table, srcs, dsts, zeros2d, zeros1d, ones1d)
    return kern(table, srcs, dsts, zeros2d)


BM = 1600
HB = HM // BM


def _mm1_body(x_ref, wl_ref, wr_ref, bl_ref, a_ref, b_ref):
    xb = x_ref[...]
    a = jnp.dot(xb, wl_ref[...], preferred_element_type=jnp.float32)
    b = jnp.dot(xb, wr_ref[...], preferred_element_type=jnp.float32) + bl_ref[...]
    a_ref[0] = a[:, :128]
    a_ref[1] = a[:, 128:]
    b_ref[0] = b[:, :128]
    b_ref[1] = b[:, 128:]


def _mm1(x, wl, wr, bl, h):
    return pl.pallas_call(
        _mm1_body,
        grid=(HB,),
        in_specs=[
            pl.BlockSpec((BM, D), lambda i: (i + h * HB, 0)),
            pl.BlockSpec((D, O), lambda i: (0, 0)),
            pl.BlockSpec((D, O), lambda i: (0, 0)),
            pl.BlockSpec((1, O), lambda i: (0, 0)),
        ],
        out_specs=[
            pl.BlockSpec((2, BM, 128), lambda i: (0, i, 0)),
            pl.BlockSpec((2, BM, 128), lambda i: (0, i, 0)),
        ],
        out_shape=[
            jax.ShapeDtypeStruct((2, HM, 128), jnp.float32),
            jax.ShapeDtypeStruct((2, HM, 128), jnp.float32),
        ],
    )(x, wl, wr, bl)


def _mm2_body(s_ref, b1_ref, deg_ref, wl_ref, wr_ref, bl_ref, a_ref, b_ref):
    inv = 1.0 / jnp.maximum(deg_ref[...], 1.0)
    h_lo = jnp.maximum(s_ref[0] * inv + b1_ref[0], 0.0)
    h_hi = jnp.maximum(s_ref[1] * inv + b1_ref[1], 0.0)
    a = (jnp.dot(h_lo, wl_ref[0], preferred_element_type=jnp.float32)
         + jnp.dot(h_hi, wl_ref[1], preferred_element_type=jnp.float32))
    b = (jnp.dot(h_lo, wr_ref[0], preferred_element_type=jnp.float32)
         + jnp.dot(h_hi, wr_ref[1], preferred_element_type=jnp.float32)
         + bl_ref[...])
    a_ref[0] = a[:, :128]
    a_ref[1] = a[:, 128:]
    b_ref[0] = b[:, :128]
    b_ref[1] = b[:, 128:]


def _mm2(s1, b1, deg, wl2s, wr2s, bl2):
    return pl.pallas_call(
        _mm2_body,
        grid=(HB,),
        in_specs=[
            pl.BlockSpec((2, BM, 128), lambda i: (0, i, 0)),
            pl.BlockSpec((2, BM, 128), lambda i: (0, i, 0)),
            pl.BlockSpec((BM, 1), lambda i: (i, 0)),
            pl.BlockSpec((2, 128, O), lambda i: (0, 0, 0)),
            pl.BlockSpec((2, 128, O), lambda i: (0, 0, 0)),
            pl.BlockSpec((1, O), lambda i: (0, 0)),
        ],
        out_specs=[
            pl.BlockSpec((2, BM, 128), lambda i: (0, i, 0)),
            pl.BlockSpec((2, BM, 128), lambda i: (0, i, 0)),
        ],
        out_shape=[
            jax.ShapeDtypeStruct((2, HM, 128), jnp.float32),
            jax.ShapeDtypeStruct((2, HM, 128), jnp.float32),
        ],
    )(s1, b1, deg, wl2s, wr2s, bl2)


BN = 2000


def _pool_body(s_ref, b_ref, deg_ref, o_ref):
    j = pl.program_id(1)
    inv = 1.0 / jnp.maximum(deg_ref[0], 1.0)
    v0 = jnp.max(s_ref[0, 0] * inv + b_ref[0, 0], axis=0, keepdims=True)
    v1 = jnp.max(s_ref[1, 0] * inv + b_ref[1, 0], axis=0, keepdims=True)
    val = jnp.concatenate([v0, v1], axis=0)

    @pl.when(j == 0)
    def _():
        o_ref[0] = val

    @pl.when(j > 0)
    def _():
        o_ref[0] = jnp.maximum(o_ref[0], val)


def _pool(s2, b2, deg):
    grid = (TSTEPS, N // BN)
    return pl.pallas_call(
        _pool_body,
        grid=grid,
        in_specs=[
            pl.BlockSpec((2, 1, BN, 128), lambda t, j: (0, t, j, 0)),
            pl.BlockSpec((2, 1, BN, 128), lambda t, j: (0, t, j, 0)),
            pl.BlockSpec((1, BN, 1), lambda t, j: (t, j, 0)),
        ],
        out_specs=pl.BlockSpec((1, 2, 128), lambda t, j: (t, 0, 0)),
        out_shape=jax.ShapeDtypeStruct((TSTEPS, 2, 128), jnp.float32),
    )(s2, b2, deg)


def _head_body(z_ref, wih_ref, whh_ref, bih_ref, bhh_ref,
               out_ref, nce_ref, acc_ref):
    z = z_ref[...]
    h = jnp.zeros((1, O), jnp.float32)
    outs = []
    for t in range(T):
        zt = z[t:t + 1, :]
        gi = jnp.dot(zt, wih_ref[...], preferred_element_type=jnp.float32) + bih_ref[...]
        gh = jnp.dot(h, whh_ref[...], preferred_element_type=jnp.float32) + bhh_ref[...]
        r = jax.nn.sigmoid(gi[:, :O] + gh[:, :O])
        zz = jax.nn.sigmoid(gi[:, O:2 * O] + gh[:, O:2 * O])
        n = jnp.tanh(gi[:, 2 * O:] + r * gh[:, 2 * O:])
        h = (1.0 - zz) * n + zz * h
        outs.append(h)
    out_ref[...] = jnp.concatenate(outs, axis=0)

    nce = jnp.float32(0.0)
    correct = jnp.float32(0.0)
    for ts in range(2, 10):
        c_t = outs[ts]
        cn = jnp.maximum(jnp.sqrt(jnp.sum(c_t * c_t)), 1e-8)
        for i in (1, 2):
            tot = []
            for ridx in (ts + i, ts + i + 2, ts + i + 3, ts + i + 4):
                a = z[ridx:ridx + 1, :]
                an = jnp.maximum(jnp.sqrt(jnp.sum(a * a)), 1e-8)
                tot.append(jnp.sum(a * c_t) / (an * cn))
            m = jnp.maximum(jnp.maximum(tot[0], tot[1]),
                            jnp.maximum(tot[2], tot[3]))
            lse = m + jnp.log(jnp.exp(tot[0] - m) + jnp.exp(tot[1] - m)
                              + jnp.exp(tot[2] - m) + jnp.exp(tot[3] - m))
            nce = nce + (tot[0] - lse)
            others = jnp.maximum(tot[1], jnp.maximum(tot[2], tot[3]))
            correct = correct + jnp.where(tot[0] >= others, 1.0, 0.0)
    nce_ref[...] = jnp.full((1, 1), nce / jnp.float32(-16.0), jnp.float32)
    acc_ref[...] = jnp.full((1, 1), correct / jnp.float32(16.0), jnp.float32)


def _head(z, wihT, whhT, bih, bhh):
    return pl.pallas_call(
        _head_body,
        in_specs=[
            pl.BlockSpec((T, O), lambda: (0, 0)),
            pl.BlockSpec((O, 3 * O), lambda: (0, 0)),
            pl.BlockSpec((O, 3 * O), lambda: (0, 0)),
            pl.BlockSpec((1, 3 * O), lambda: (0, 0)),
            pl.BlockSpec((1, 3 * O), lambda: (0, 0)),
        ],
        out_specs=[
            pl.BlockSpec((T, O), lambda: (0, 0)),
            pl.BlockSpec((1, 1), lambda: (0, 0)),
            pl.BlockSpec((1, 1), lambda: (0, 0)),
        ],
        out_shape=[
            jax.ShapeDtypeStruct((T, O), jnp.float32),
            jax.ShapeDtypeStruct((1, 1), jnp.float32),
            jax.ShapeDtypeStruct((1, 1), jnp.float32),
        ],
    )(z, wihT, whhT, bih, bhh)


def kernel(x, edge_index, batch, W_l1, b_l1, W_r1, W_l2, b_l2, W_r2,
           W_ih, W_hh, b_ih, b_hh):
    del batch
    xf = x.reshape(M, D)

    src = (edge_index[:, 0, :]
           + ((jnp.arange(T, dtype=jnp.int32) % TSTEPS) * N)[:, None])
    srcs = jnp.stack([src, src + HM]).reshape(2, T, NSUB, NGROUP, NBUF, CH)
    dsts = edge_index[:, 1, :].reshape(T, NSUB, NGROUP, NBUF, CH)

    wl2s = W_l2.reshape(2, 128, O)
    wr2s = W_r2.reshape(2, 128, O)
    bl1 = b_l1.reshape(1, O)
    bl2 = b_l2.reshape(1, O)

    pooled = []
    for h in (0, 1):
        a1, b1 = _mm1(xf, W_l1, W_r1, bl1, h)
        s1, deg = _sc_segsum(a1.reshape(2 * HM, 128), srcs, dsts,
                             h * TSTEPS, with_deg=True)
        a2, b2 = _mm2(s1.reshape(2, HM, 128), b1, deg.reshape(HM, 1),
                      wl2s, wr2s, bl2)
        s2 = _sc_segsum(a2.reshape(2 * HM, 128), srcs, dsts,
                        h * TSTEPS, with_deg=False)
        pooled.append(_pool(s2, b2.reshape(2, TSTEPS, N, 128),
                            deg.reshape(TSTEPS, N, 1)))
    z = jnp.concatenate([p.reshape(TSTEPS, O) for p in pooled], axis=0)

    gru_out, nce, acc = _head(z, W_ih.T, W_hh.T,
                              b_ih.reshape(1, 3 * O), b_hh.reshape(1, 3 * O))
    return nce[0, 0], acc[0, 0], gru_out[None]


# speedup vs baseline: 6.1427x; 1.0058x over previous
import functools

import jax
import jax.numpy as jnp
from jax import lax
from jax.experimental import pallas as pl
from jax.experimental.pallas import tpu as pltpu
from jax.experimental.pallas import tpu_sc as plsc

T = 16
N = 10000
E = 160000
D = 256
O = 256
NCORES = 2
NSUB = 16
EPT = E // NSUB
CH = 100
NCHUNK = EPT // CH
NGROUP = 50
ROW_CH = 640
ROW_TAIL = N - ROW_CH * (NSUB - 1)
DEG_CH = 640
DEG_TAIL = N - DEG_CH * (NSUB - 1)
NBUF = 2
ZB = 40
M = T * N
TSTEPS = T // 2
HM = TSTEPS * N


def _sc_segsum_body(with_deg, t0, *refs):
    if with_deg:
        (table, srcs, dsts, zeros2d, zeros1d, ones1d, s_out, deg_out,
         srcr, dstr, rows_v, ones_v, deg_v, dego_v, zv_v, acc, deg_acc,
         *sems) = refs
    else:
        (table, srcs, dsts, zeros2d, s_out,
         srcr, dstr, rows_v, ones_v, deg_v, dego_v, zv_v, acc, deg_acc,
         *sems) = refs
    gsem = sems[:NBUF]
    ssem = sems[NBUF:2 * NBUF]
    dsem = sems[2 * NBUF:3 * NBUF]
    isem0, isem1, csem = sems[3 * NBUF:3 * NBUF + 3]
    c = lax.axis_index("c")
    s = lax.axis_index("s")
    last = s == NSUB - 1
    row_off = pl.multiple_of(s * ROW_CH, 8)
    deg_off = pl.multiple_of(s * DEG_CH, 8)

    pltpu.sync_copy(zeros2d, zv_v)
    if with_deg:
        pltpu.sync_copy(ones1d, ones_v)
        pltpu.sync_copy(zeros1d, deg_v)

    def _zero():
        @pl.when(jnp.logical_not(last))
        def _():
            for i in range(ROW_CH // ZB):
                pltpu.sync_copy(zv_v, acc.at[pl.ds(row_off + i * ZB, ZB)])
            if with_deg:
                pltpu.sync_copy(deg_v, deg_acc.at[pl.ds(deg_off, DEG_CH)])

        @pl.when(last)
        def _():
            for i in range(ROW_TAIL // ZB):
                pltpu.sync_copy(zv_v, acc.at[pl.ds(ROW_CH * (NSUB - 1)
                                                   + i * ZB, ZB)])
            if with_deg:
                pltpu.sync_copy(deg_v.at[pl.ds(0, DEG_TAIL)],
                                deg_acc.at[pl.ds(DEG_CH * (NSUB - 1), DEG_TAIL)])

    def _prologue(ta):
        pltpu.sync_copy(srcs.at[c, ta, s, 0], srcr.at[0])
        pltpu.sync_copy(dsts.at[ta, s, 0], dstr.at[0])
        pltpu.async_copy(srcs.at[c, ta, s, 1], srcr.at[1], isem0)
        pltpu.async_copy(dsts.at[ta, s, 1], dstr.at[1], isem1)
        for b in range(NBUF):
            pltpu.async_copy(table.at[srcr.at[0, b]], rows_v.at[b], gsem[b])

    _zero()
    plsc.subcore_barrier()
    _prologue(t0)

    @pl.loop(0, TSTEPS)
    def _t(t):
        ta = t0 + t

        @pl.loop(0, NGROUP - 1)
        def _grp(k):
            p = lax.rem(k, 2)
            pn = 1 - p
            pltpu.make_async_copy(srcs.at[c, ta, s, 0], srcr.at[pn],
                                  isem0).wait()
            pltpu.make_async_copy(dsts.at[ta, s, 0], dstr.at[pn],
                                  isem1).wait()
            for b in range(NBUF):
                pltpu.make_async_copy(table.at[srcr.at[p, b]], rows_v.at[b],
                                      gsem[b]).wait()
                sd = pltpu.async_copy(rows_v.at[b], acc.at[dstr.at[p, b]],
                                      ssem[b], add=True)
                if with_deg:
                    dd = pltpu.async_copy(ones_v, deg_acc.at[dstr.at[p, b]],
                                          dsem[b], add=True)
                sd.wait()
                if with_deg:
                    dd.wait()
                pltpu.async_copy(table.at[srcr.at[pn, b]], rows_v.at[b],
                                 gsem[b])
            @pl.when(k + 2 < NGROUP)
            def _():
                pltpu.async_copy(srcs.at[c, ta, s, k + 2], srcr.at[p], isem0)
                pltpu.async_copy(dsts.at[ta, s, k + 2], dstr.at[p], isem1)

        pf = (NGROUP - 1) % 2
        for b in range(NBUF):
            pltpu.make_async_copy(table.at[srcr.at[pf, b]], rows_v.at[b],
                                  gsem[b]).wait()
            pltpu.sync_copy(rows_v.at[b], acc.at[dstr.at[pf, b]], add=True)
            if with_deg:
                pltpu.sync_copy(ones_v, deg_acc.at[dstr.at[pf, b]], add=True)
        plsc.subcore_barrier()

        @pl.when(jnp.logical_not(last))
        def _():
            pltpu.async_copy(acc.at[pl.ds(row_off, ROW_CH)],
                             s_out.at[c, t, pl.ds(row_off, ROW_CH)], csem)
            if with_deg:
                @pl.when(c == 0)
                def _():
                    pltpu.sync_copy(deg_acc.at[pl.ds(deg_off, DEG_CH)], dego_v)
                    pltpu.sync_copy(
                        dego_v,
                        deg_out.at[pl.ds(pl.multiple_of(t * N + deg_off, 8),
                                         DEG_CH)])

        @pl.when(last)
        def _():
            pltpu.async_copy(acc.at[pl.ds(ROW_CH * (NSUB - 1), ROW_TAIL)],
                             s_out.at[c, t, pl.ds(ROW_CH * (NSUB - 1),
                                                  ROW_TAIL)], csem)
            if with_deg:
                @pl.when(c == 0)
                def _():
                    pltpu.sync_copy(deg_acc.at[pl.ds(DEG_CH * (NSUB - 1),
                                                     DEG_TAIL)],
                                    dego_v.at[pl.ds(0, DEG_TAIL)])
                    pltpu.sync_copy(dego_v.at[pl.ds(0, DEG_TAIL)],
                                    deg_out.at[pl.ds(pl.multiple_of(
                                        t * N + DEG_CH * (NSUB - 1), 8),
                                        DEG_TAIL)])

        @pl.when(t + 1 < TSTEPS)
        def _():
            _prologue(ta + 1)

        @pl.when(jnp.logical_not(last))
        def _():
            pltpu.make_async_copy(acc.at[pl.ds(row_off, ROW_CH)],
                                  s_out.at[c, t, pl.ds(row_off, ROW_CH)],
                                  csem).wait()

        @pl.when(last)
        def _():
            pltpu.make_async_copy(acc.at[pl.ds(ROW_CH * (NSUB - 1), ROW_TAIL)],
                                  s_out.at[c, t, pl.ds(ROW_CH * (NSUB - 1),
                                                       ROW_TAIL)], csem).wait()

        @pl.when(t + 1 < TSTEPS)
        def _():
            _zero()
        plsc.subcore_barrier()


def _sc_segsum(table, srcs, dsts, t0, with_deg):
    zeros2d = jnp.zeros((ZB, 128), jnp.float32)
    zeros1d = jnp.zeros((DEG_CH,), jnp.float32)
    ones1d = jnp.ones((CH,), jnp.float32)
    out_type = [jax.ShapeDtypeStruct((NCORES, TSTEPS, N, 128), jnp.float32)]
    if with_deg:
        out_type.append(jax.ShapeDtypeStruct((TSTEPS * N,), jnp.float32))
    mesh = plsc.VectorSubcoreMesh(core_axis_name="c", subcore_axis_name="s",
                                  num_cores=NCORES, num_subcores=NSUB)
    scratch = [
        pltpu.VMEM((2, NBUF, CH), jnp.int32),
        pltpu.VMEM((2, NBUF, CH), jnp.int32),
        pltpu.VMEM((NBUF, CH, 128), jnp.float32),
        pltpu.VMEM((CH,), jnp.float32),
        pltpu.VMEM((DEG_CH,), jnp.float32),
        pltpu.VMEM((DEG_CH,), jnp.float32),
        pltpu.VMEM((ZB, 128), jnp.float32),
        pltpu.VMEM_SHARED((N, 128), jnp.float32),
        pltpu.VMEM_SHARED((N,), jnp.float32),
    ] + [pltpu.SemaphoreType.DMA] * (3 * NBUF + 3)
    kern = pl.kernel(
        functools.partial(_sc_segsum_body, with_deg, t0),
        out_type=tuple(out_type) if with_deg else out_type[0],
        mesh=mesh,
        scratch_types=scratch,
    )
    if with_deg:
        return kern(table, srcs, dsts, zeros2d, zeros1d, ones1d)
    return kern(table, srcs, dsts, zeros2d)


BM = 1600
HB = HM // BM


def _mm1_body(x_ref, wl_ref, wr_ref, bl_ref, a_ref, b_ref):
    xb = x_ref[...]
    a = jnp.dot(xb, wl_ref[...], preferred_element_type=jnp.float32)
    b = jnp.dot(xb, wr_ref[...], preferred_element_type=jnp.float32) + bl_ref[...]
    a_ref[0] = a[:, :128]
    a_ref[1] = a[:, 128:]
    b_ref[0] = b[:, :128]
    b_ref[1] = b[:, 128:]


def _mm1(x, wl, wr, bl, h):
    return pl.pallas_call(
        _mm1_body,
        grid=(HB,),
        in_specs=[
            pl.BlockSpec((BM, D), lambda i: (i + h * HB, 0)),
            pl.BlockSpec((D, O), lambda i: (0, 0)),
            pl.BlockSpec((D, O), lambda i: (0, 0)),
            pl.BlockSpec((1, O), lambda i: (0, 0)),
        ],
        out_specs=[
            pl.BlockSpec((2, BM, 128), lambda i: (0, i, 0)),
            pl.BlockSpec((2, BM, 128), lambda i: (0, i, 0)),
        ],
        out_shape=[
            jax.ShapeDtypeStruct((2, HM, 128), jnp.float32),
            jax.ShapeDtypeStruct((2, HM, 128), jnp.float32),
        ],
    )(x, wl, wr, bl)


def _mm2_body(s_ref, b1_ref, deg_ref, wl_ref, wr_ref, bl_ref, a_ref, b_ref):
    inv = 1.0 / jnp.maximum(deg_ref[...], 1.0)
    h_lo = jnp.maximum(s_ref[0] * inv + b1_ref[0], 0.0)
    h_hi = jnp.maximum(s_ref[1] * inv + b1_ref[1], 0.0)
    a = (jnp.dot(h_lo, wl_ref[0], preferred_element_type=jnp.float32)
         + jnp.dot(h_hi, wl_ref[1], preferred_element_type=jnp.float32))
    b = (jnp.dot(h_lo, wr_ref[0], preferred_element_type=jnp.float32)
         + jnp.dot(h_hi, wr_ref[1], preferred_element_type=jnp.float32)
         + bl_ref[...])
    a_ref[0] = a[:, :128]
    a_ref[1] = a[:, 128:]
    b_ref[0] = b[:, :128]
    b_ref[1] = b[:, 128:]


def _mm2(s1, b1, deg, wl2s, wr2s, bl2):
    return pl.pallas_call(
        _mm2_body,
        grid=(HB,),
        in_specs=[
            pl.BlockSpec((2, BM, 128), lambda i: (0, i, 0)),
            pl.BlockSpec((2, BM, 128), lambda i: (0, i, 0)),
            pl.BlockSpec((BM, 1), lambda i: (i, 0)),
            pl.BlockSpec((2, 128, O), lambda i: (0, 0, 0)),
            pl.BlockSpec((2, 128, O), lambda i: (0, 0, 0)),
            pl.BlockSpec((1, O), lambda i: (0, 0)),
        ],
        out_specs=[
            pl.BlockSpec((2, BM, 128), lambda i: (0, i, 0)),
            pl.BlockSpec((2, BM, 128), lambda i: (0, i, 0)),
        ],
        out_shape=[
            jax.ShapeDtypeStruct((2, HM, 128), jnp.float32),
            jax.ShapeDtypeStruct((2, HM, 128), jnp.float32),
        ],
    )(s1, b1, deg, wl2s, wr2s, bl2)


BN = 2000


def _pool_body(s_ref, b_ref, deg_ref, o_ref):
    j = pl.program_id(1)
    inv = 1.0 / jnp.maximum(deg_ref[0], 1.0)
    v0 = jnp.max(s_ref[0, 0] * inv + b_ref[0, 0], axis=0, keepdims=True)
    v1 = jnp.max(s_ref[1, 0] * inv + b_ref[1, 0], axis=0, keepdims=True)
    val = jnp.concatenate([v0, v1], axis=0)

    @pl.when(j == 0)
    def _():
        o_ref[0] = val

    @pl.when(j > 0)
    def _():
        o_ref[0] = jnp.maximum(o_ref[0], val)


def _pool(s2, b2, deg):
    grid = (TSTEPS, N // BN)
    return pl.pallas_call(
        _pool_body,
        grid=grid,
        in_specs=[
            pl.BlockSpec((2, 1, BN, 128), lambda t, j: (0, t, j, 0)),
            pl.BlockSpec((2, 1, BN, 128), lambda t, j: (0, t, j, 0)),
            pl.BlockSpec((1, BN, 1), lambda t, j: (t, j, 0)),
        ],
        out_specs=pl.BlockSpec((1, 2, 128), lambda t, j: (t, 0, 0)),
        out_shape=jax.ShapeDtypeStruct((TSTEPS, 2, 128), jnp.float32),
    )(s2, b2, deg)


def _head_body(z_ref, wih_ref, whh_ref, bih_ref, bhh_ref,
               out_ref, nce_ref, acc_ref):
    z = z_ref[...]
    h = jnp.zeros((1, O), jnp.float32)
    outs = []
    for t in range(T):
        zt = z[t:t + 1, :]
        gi = jnp.dot(zt, wih_ref[...], preferred_element_type=jnp.float32) + bih_ref[...]
        gh = jnp.dot(h, whh_ref[...], preferred_element_type=jnp.float32) + bhh_ref[...]
        r = jax.nn.sigmoid(gi[:, :O] + gh[:, :O])
        zz = jax.nn.sigmoid(gi[:, O:2 * O] + gh[:, O:2 * O])
        n = jnp.tanh(gi[:, 2 * O:] + r * gh[:, 2 * O:])
        h = (1.0 - zz) * n + zz * h
        outs.append(h)
    out_ref[...] = jnp.concatenate(outs, axis=0)

    nce = jnp.float32(0.0)
    correct = jnp.float32(0.0)
    for ts in range(2, 10):
        c_t = outs[ts]
        cn = jnp.maximum(jnp.sqrt(jnp.sum(c_t * c_t)), 1e-8)
        for i in (1, 2):
            tot = []
            for ridx in (ts + i, ts + i + 2, ts + i + 3, ts + i + 4):
                a = z[ridx:ridx + 1, :]
                an = jnp.maximum(jnp.sqrt(jnp.sum(a * a)), 1e-8)
                tot.append(jnp.sum(a * c_t) / (an * cn))
            m = jnp.maximum(jnp.maximum(tot[0], tot[1]),
                            jnp.maximum(tot[2], tot[3]))
            lse = m + jnp.log(jnp.exp(tot[0] - m) + jnp.exp(tot[1] - m)
                              + jnp.exp(tot[2] - m) + jnp.exp(tot[3] - m))
            nce = nce + (tot[0] - lse)
            others = jnp.maximum(tot[1], jnp.maximum(tot[2], tot[3]))
            correct = correct + jnp.where(tot[0] >= others, 1.0, 0.0)
    nce_ref[...] = jnp.full((1, 1), nce / jnp.float32(-16.0), jnp.float32)
    acc_ref[...] = jnp.full((1, 1), correct / jnp.float32(16.0), jnp.float32)


def _head(z, wihT, whhT, bih, bhh):
    return pl.pallas_call(
        _head_body,
        in_specs=[
            pl.BlockSpec((T, O), lambda: (0, 0)),
            pl.BlockSpec((O, 3 * O), lambda: (0, 0)),
            pl.BlockSpec((O, 3 * O), lambda: (0, 0)),
            pl.BlockSpec((1, 3 * O), lambda: (0, 0)),
            pl.BlockSpec((1, 3 * O), lambda: (0, 0)),
        ],
        out_specs=[
            pl.BlockSpec((T, O), lambda: (0, 0)),
            pl.BlockSpec((1, 1), lambda: (0, 0)),
            pl.BlockSpec((1, 1), lambda: (0, 0)),
        ],
        out_shape=[
            jax.ShapeDtypeStruct((T, O), jnp.float32),
            jax.ShapeDtypeStruct((1, 1), jnp.float32),
            jax.ShapeDtypeStruct((1, 1), jnp.float32),
        ],
    )(z, wihT, whhT, bih, bhh)


def kernel(x, edge_index, batch, W_l1, b_l1, W_r1, W_l2, b_l2, W_r2,
           W_ih, W_hh, b_ih, b_hh):
    del batch
    xf = x.reshape(M, D)

    src = (edge_index[:, 0, :]
           + ((jnp.arange(T, dtype=jnp.int32) % TSTEPS) * N)[:, None])
    srcs = jnp.stack([src, src + HM]).reshape(2, T, NSUB, NGROUP, NBUF, CH)
    dsts = edge_index[:, 1, :].reshape(T, NSUB, NGROUP, NBUF, CH)

    wl2s = W_l2.reshape(2, 128, O)
    wr2s = W_r2.reshape(2, 128, O)
    bl1 = b_l1.reshape(1, O)
    bl2 = b_l2.reshape(1, O)

    pooled = []
    for h in (0, 1):
        a1, b1 = _mm1(xf, W_l1, W_r1, bl1, h)
        s1, deg = _sc_segsum(a1.reshape(2 * HM, 128), srcs, dsts,
                             h * TSTEPS, with_deg=True)
        a2, b2 = _mm2(s1.reshape(2, HM, 128), b1, deg.reshape(HM, 1),
                      wl2s, wr2s, bl2)
        s2 = _sc_segsum(a2.reshape(2 * HM, 128), srcs, dsts,
                        h * TSTEPS, with_deg=False)
        pooled.append(_pool(s2, b2.reshape(2, TSTEPS, N, 128),
                            deg.reshape(TSTEPS, N, 1)))
    z = jnp.concatenate([p.reshape(TSTEPS, O) for p in pooled], axis=0)

    gru_out, nce, acc = _head(z, W_ih.T, W_hh.T,
                              b_ih.reshape(1, 3 * O), b_hh.reshape(1, 3 * O))
    return nce[0, 0], acc[0, 0], gru_out[None]


# speedup vs baseline: 6.3974x; 1.0415x over previous
import functools

import jax
import jax.numpy as jnp
from jax import lax
from jax.experimental import pallas as pl
from jax.experimental.pallas import tpu as pltpu
from jax.experimental.pallas import tpu_sc as plsc

T = 16
N = 10000
E = 160000
D = 256
O = 256
NCORES = 2
NSUB = 16
EPT = E // NSUB
CH = 125
NCHUNK = EPT // CH
NGROUP = 40
ROW_CH = 640
ROW_TAIL = N - ROW_CH * (NSUB - 1)
DEG_CH = 640
DEG_TAIL = N - DEG_CH * (NSUB - 1)
NBUF = 2
ZB = 40
M = T * N
TSTEPS = T // 2
HM = TSTEPS * N


def _sc_segsum_body(with_deg, t0, *refs):
    if with_deg:
        (table, srcs, dsts, zeros2d, zeros1d, ones1d, s_out, deg_out,
         srcr, dstr, rows_v, ones_v, deg_v, dego_v, zv_v, acc, deg_acc,
         *sems) = refs
    else:
        (table, srcs, dsts, zeros2d, s_out,
         srcr, dstr, rows_v, ones_v, deg_v, dego_v, zv_v, acc, deg_acc,
         *sems) = refs
    gsem = sems[:NBUF]
    ssem = sems[NBUF:2 * NBUF]
    dsem = sems[2 * NBUF:3 * NBUF]
    isem0, isem1, csem = sems[3 * NBUF:3 * NBUF + 3]
    c = lax.axis_index("c")
    s = lax.axis_index("s")
    last = s == NSUB - 1
    row_off = pl.multiple_of(s * ROW_CH, 8)
    deg_off = pl.multiple_of(s * DEG_CH, 8)

    pltpu.sync_copy(zeros2d, zv_v)
    if with_deg:
        pltpu.sync_copy(ones1d, ones_v)
        pltpu.sync_copy(zeros1d, deg_v)

    def _zero():
        @pl.when(jnp.logical_not(last))
        def _():
            for i in range(ROW_CH // ZB):
                pltpu.sync_copy(zv_v, acc.at[pl.ds(row_off + i * ZB, ZB)])
            if with_deg:
                pltpu.sync_copy(deg_v, deg_acc.at[pl.ds(deg_off, DEG_CH)])

        @pl.when(last)
        def _():
            for i in range(ROW_TAIL // ZB):
                pltpu.sync_copy(zv_v, acc.at[pl.ds(ROW_CH * (NSUB - 1)
                                                   + i * ZB, ZB)])
            if with_deg:
                pltpu.sync_copy(deg_v.at[pl.ds(0, DEG_TAIL)],
                                deg_acc.at[pl.ds(DEG_CH * (NSUB - 1), DEG_TAIL)])

    def _prologue(ta):
        pltpu.sync_copy(srcs.at[c, ta, s, 0], srcr.at[0])
        pltpu.sync_copy(dsts.at[ta, s, 0], dstr.at[0])
        pltpu.async_copy(srcs.at[c, ta, s, 1], srcr.at[1], isem0)
        pltpu.async_copy(dsts.at[ta, s, 1], dstr.at[1], isem1)
        for b in range(NBUF):
            pltpu.async_copy(table.at[srcr.at[0, b]], rows_v.at[b], gsem[b])

    _zero()
    plsc.subcore_barrier()
    _prologue(t0)

    @pl.loop(0, TSTEPS)
    def _t(t):
        ta = t0 + t

        @pl.loop(0, NGROUP - 1)
        def _grp(k):
            p = lax.rem(k, 2)
            pn = 1 - p
            pltpu.make_async_copy(srcs.at[c, ta, s, 0], srcr.at[pn],
                                  isem0).wait()
            pltpu.make_async_copy(dsts.at[ta, s, 0], dstr.at[pn],
                                  isem1).wait()
            for b in range(NBUF):
                pltpu.make_async_copy(table.at[srcr.at[p, b]], rows_v.at[b],
                                      gsem[b]).wait()
                sd = pltpu.async_copy(rows_v.at[b], acc.at[dstr.at[p, b]],
                                      ssem[b], add=True)
                if with_deg:
                    dd = pltpu.async_copy(ones_v, deg_acc.at[dstr.at[p, b]],
                                          dsem[b], add=True)
                sd.wait()
                if with_deg:
                    dd.wait()
                pltpu.async_copy(table.at[srcr.at[pn, b]], rows_v.at[b],
                                 gsem[b])
            @pl.when(k + 2 < NGROUP)
            def _():
                pltpu.async_copy(srcs.at[c, ta, s, k + 2], srcr.at[p], isem0)
                pltpu.async_copy(dsts.at[ta, s, k + 2], dstr.at[p], isem1)

        pf = (NGROUP - 1) % 2
        for b in range(NBUF):
            pltpu.make_async_copy(table.at[srcr.at[pf, b]], rows_v.at[b],
                                  gsem[b]).wait()
            pltpu.sync_copy(rows_v.at[b], acc.at[dstr.at[pf, b]], add=True)
            if with_deg:
                pltpu.sync_copy(ones_v, deg_acc.at[dstr.at[pf, b]], add=True)
        plsc.subcore_barrier()

        @pl.when(jnp.logical_not(last))
        def _():
            pltpu.async_copy(acc.at[pl.ds(row_off, ROW_CH)],
                             s_out.at[c, t, pl.ds(row_off, ROW_CH)], csem)
            if with_deg:
                @pl.when(c == 0)
                def _():
                    pltpu.sync_copy(deg_acc.at[pl.ds(deg_off, DEG_CH)], dego_v)
                    pltpu.sync_copy(
                        dego_v,
                        deg_out.at[pl.ds(pl.multiple_of(t * N + deg_off, 8),
                                         DEG_CH)])

        @pl.when(last)
        def _():
            pltpu.async_copy(acc.at[pl.ds(ROW_CH * (NSUB - 1), ROW_TAIL)],
                             s_out.at[c, t, pl.ds(ROW_CH * (NSUB - 1),
                                                  ROW_TAIL)], csem)
            if with_deg:
                @pl.when(c == 0)
                def _():
                    pltpu.sync_copy(deg_acc.at[pl.ds(DEG_CH * (NSUB - 1),
                                                     DEG_TAIL)],
                                    dego_v.at[pl.ds(0, DEG_TAIL)])
                    pltpu.sync_copy(dego_v.at[pl.ds(0, DEG_TAIL)],
                                    deg_out.at[pl.ds(pl.multiple_of(
                                        t * N + DEG_CH * (NSUB - 1), 8),
                                        DEG_TAIL)])

        @pl.when(t + 1 < TSTEPS)
        def _():
            _prologue(ta + 1)

        @pl.when(jnp.logical_not(last))
        def _():
            pltpu.make_async_copy(acc.at[pl.ds(row_off, ROW_CH)],
                                  s_out.at[c, t, pl.ds(row_off, ROW_CH)],
                                  csem).wait()

        @pl.when(last)
        def _():
            pltpu.make_async_copy(acc.at[pl.ds(ROW_CH * (NSUB - 1), ROW_TAIL)],
                                  s_out.at[c, t, pl.ds(ROW_CH * (NSUB - 1),
                                                       ROW_TAIL)], csem).wait()

        @pl.when(t + 1 < TSTEPS)
        def _():
            _zero()
        plsc.subcore_barrier()


def _sc_segsum(table, srcs, dsts, t0, with_deg):
    zeros2d = jnp.zeros((ZB, 128), jnp.float32)
    zeros1d = jnp.zeros((DEG_CH,), jnp.float32)
    ones1d = jnp.ones((CH,), jnp.float32)
    out_type = [jax.ShapeDtypeStruct((NCORES, TSTEPS, N, 128), jnp.float32)]
    if with_deg:
        out_type.append(jax.ShapeDtypeStruct((TSTEPS * N,), jnp.float32))
    mesh = plsc.VectorSubcoreMesh(core_axis_name="c", subcore_axis_name="s",
                                  num_cores=NCORES, num_subcores=NSUB)
    scratch = [
        pltpu.VMEM((2, NBUF, CH), jnp.int32),
        pltpu.VMEM((2, NBUF, CH), jnp.int32),
        pltpu.VMEM((NBUF, CH, 128), jnp.float32),
        pltpu.VMEM((CH,), jnp.float32),
        pltpu.VMEM((DEG_CH,), jnp.float32),
        pltpu.VMEM((DEG_CH,), jnp.float32),
        pltpu.VMEM((ZB, 128), jnp.float32),
        pltpu.VMEM_SHARED((N, 128), jnp.float32),
        pltpu.VMEM_SHARED((N,), jnp.float32),
    ] + [pltpu.SemaphoreType.DMA] * (3 * NBUF + 3)
    kern = pl.kernel(
        functools.partial(_sc_segsum_body, with_deg, t0),
        out_type=tuple(out_type) if with_deg else out_type[0],
        mesh=mesh,
        scratch_types=scratch,
    )
    if with_deg:
        return kern(table, srcs, dsts, zeros2d, zeros1d, ones1d)
    return kern(table, srcs, dsts, zeros2d)


BM = 1600
HB = HM // BM


def _mm1_body(x_ref, wl_ref, wr_ref, bl_ref, a_ref, b_ref):
    xb = x_ref[...]
    a = jnp.dot(xb, wl_ref[...], preferred_element_type=jnp.float32)
    b = jnp.dot(xb, wr_ref[...], preferred_element_type=jnp.float32) + bl_ref[...]
    a_ref[0] = a[:, :128]
    a_ref[1] = a[:, 128:]
    b_ref[0] = b[:, :128]
    b_ref[1] = b[:, 128:]


def _mm1(x, wl, wr, bl, h):
    return pl.pallas_call(
        _mm1_body,
        grid=(HB,),
        in_specs=[
            pl.BlockSpec((BM, D), lambda i: (i + h * HB, 0)),
            pl.BlockSpec((D, O), lambda i: (0, 0)),
            pl.BlockSpec((D, O), lambda i: (0, 0)),
            pl.BlockSpec((1, O), lambda i: (0, 0)),
        ],
        out_specs=[
            pl.BlockSpec((2, BM, 128), lambda i: (0, i, 0)),
            pl.BlockSpec((2, BM, 128), lambda i: (0, i, 0)),
        ],
        out_shape=[
            jax.ShapeDtypeStruct((2, HM, 128), jnp.float32),
            jax.ShapeDtypeStruct((2, HM, 128), jnp.float32),
        ],
    )(x, wl, wr, bl)


def _mm2_body(s_ref, b1_ref, deg_ref, wl_ref, wr_ref, bl_ref, a_ref, b_ref):
    inv = 1.0 / jnp.maximum(deg_ref[...], 1.0)
    h_lo = jnp.maximum(s_ref[0] * inv + b1_ref[0], 0.0)
    h_hi = jnp.maximum(s_ref[1] * inv + b1_ref[1], 0.0)
    a = (jnp.dot(h_lo, wl_ref[0], preferred_element_type=jnp.float32)
         + jnp.dot(h_hi, wl_ref[1], preferred_element_type=jnp.float32))
    b = (jnp.dot(h_lo, wr_ref[0], preferred_element_type=jnp.float32)
         + jnp.dot(h_hi, wr_ref[1], preferred_element_type=jnp.float32)
         + bl_ref[...])
    a_ref[0] = a[:, :128]
    a_ref[1] = a[:, 128:]
    b_ref[0] = b[:, :128]
    b_ref[1] = b[:, 128:]


def _mm2(s1, b1, deg, wl2s, wr2s, bl2):
    return pl.pallas_call(
        _mm2_body,
        grid=(HB,),
        in_specs=[
            pl.BlockSpec((2, BM, 128), lambda i: (0, i, 0)),
            pl.BlockSpec((2, BM, 128), lambda i: (0, i, 0)),
            pl.BlockSpec((BM, 1), lambda i: (i, 0)),
            pl.BlockSpec((2, 128, O), lambda i: (0, 0, 0)),
            pl.BlockSpec((2, 128, O), lambda i: (0, 0, 0)),
            pl.BlockSpec((1, O), lambda i: (0, 0)),
        ],
        out_specs=[
            pl.BlockSpec((2, BM, 128), lambda i: (0, i, 0)),
            pl.BlockSpec((2, BM, 128), lambda i: (0, i, 0)),
        ],
        out_shape=[
            jax.ShapeDtypeStruct((2, HM, 128), jnp.float32),
            jax.ShapeDtypeStruct((2, HM, 128), jnp.float32),
        ],
    )(s1, b1, deg, wl2s, wr2s, bl2)


BN = 2000


def _pool_body(s_ref, b_ref, deg_ref, o_ref):
    j = pl.program_id(1)
    inv = 1.0 / jnp.maximum(deg_ref[0], 1.0)
    v0 = jnp.max(s_ref[0, 0] * inv + b_ref[0, 0], axis=0, keepdims=True)
    v1 = jnp.max(s_ref[1, 0] * inv + b_ref[1, 0], axis=0, keepdims=True)
    val = jnp.concatenate([v0, v1], axis=0)

    @pl.when(j == 0)
    def _():
        o_ref[0] = val

    @pl.when(j > 0)
    def _():
        o_ref[0] = jnp.maximum(o_ref[0], val)


def _pool(s2, b2, deg):
    grid = (TSTEPS, N // BN)
    return pl.pallas_call(
        _pool_body,
        grid=grid,
        in_specs=[
            pl.BlockSpec((2, 1, BN, 128), lambda t, j: (0, t, j, 0)),
            pl.BlockSpec((2, 1, BN, 128), lambda t, j: (0, t, j, 0)),
            pl.BlockSpec((1, BN, 1), lambda t, j: (t, j, 0)),
        ],
        out_specs=pl.BlockSpec((1, 2, 128), lambda t, j: (t, 0, 0)),
        out_shape=jax.ShapeDtypeStruct((TSTEPS, 2, 128), jnp.float32),
    )(s2, b2, deg)


def _head_body(z_ref, wih_ref, whh_ref, bih_ref, bhh_ref,
               out_ref, nce_ref, acc_ref):
    z = z_ref[...]
    h = jnp.zeros((1, O), jnp.float32)
    outs = []
    for t in range(T):
        zt = z[t:t + 1, :]
        gi = jnp.dot(zt, wih_ref[...], preferred_element_type=jnp.float32) + bih_ref[...]
        gh = jnp.dot(h, whh_ref[...], preferred_element_type=jnp.float32) + bhh_ref[...]
        r = jax.nn.sigmoid(gi[:, :O] + gh[:, :O])
        zz = jax.nn.sigmoid(gi[:, O:2 * O] + gh[:, O:2 * O])
        n = jnp.tanh(gi[:, 2 * O:] + r * gh[:, 2 * O:])
        h = (1.0 - zz) * n + zz * h
        outs.append(h)
    out_ref[...] = jnp.concatenate(outs, axis=0)

    nce = jnp.float32(0.0)
    correct = jnp.float32(0.0)
    for ts in range(2, 10):
        c_t = outs[ts]
        cn = jnp.maximum(jnp.sqrt(jnp.sum(c_t * c_t)), 1e-8)
        for i in (1, 2):
            tot = []
            for ridx in (ts + i, ts + i + 2, ts + i + 3, ts + i + 4):
                a = z[ridx:ridx + 1, :]
                an = jnp.maximum(jnp.sqrt(jnp.sum(a * a)), 1e-8)
                tot.append(jnp.sum(a * c_t) / (an * cn))
            m = jnp.maximum(jnp.maximum(tot[0], tot[1]),
                            jnp.maximum(tot[2], tot[3]))
            lse = m + jnp.log(jnp.exp(tot[0] - m) + jnp.exp(tot[1] - m)
                              + jnp.exp(tot[2] - m) + jnp.exp(tot[3] - m))
            nce = nce + (tot[0] - lse)
            others = jnp.maximum(tot[1], jnp.maximum(tot[2], tot[3]))
            correct = correct + jnp.where(tot[0] >= others, 1.0, 0.0)
    nce_ref[...] = jnp.full((1, 1), nce / jnp.float32(-16.0), jnp.float32)
    acc_ref[...] = jnp.full((1, 1), correct / jnp.float32(16.0), jnp.float32)


def _head(z, wihT, whhT, bih, bhh):
    return pl.pallas_call(
        _head_body,
        in_specs=[
            pl.BlockSpec((T, O), lambda: (0, 0)),
            pl.BlockSpec((O, 3 * O), lambda: (0, 0)),
            pl.BlockSpec((O, 3 * O), lambda: (0, 0)),
            pl.BlockSpec((1, 3 * O), lambda: (0, 0)),
            pl.BlockSpec((1, 3 * O), lambda: (0, 0)),
        ],
        out_specs=[
            pl.BlockSpec((T, O), lambda: (0, 0)),
            pl.BlockSpec((1, 1), lambda: (0, 0)),
            pl.BlockSpec((1, 1), lambda: (0, 0)),
        ],
        out_shape=[
            jax.ShapeDtypeStruct((T, O), jnp.float32),
            jax.ShapeDtypeStruct((1, 1), jnp.float32),
            jax.ShapeDtypeStruct((1, 1), jnp.float32),
        ],
    )(z, wihT, whhT, bih, bhh)


def kernel(x, edge_index, batch, W_l1, b_l1, W_r1, W_l2, b_l2, W_r2,
           W_ih, W_hh, b_ih, b_hh):
    del batch
    xf = x.reshape(M, D)

    src = (edge_index[:, 0, :]
           + ((jnp.arange(T, dtype=jnp.int32) % TSTEPS) * N)[:, None])
    srcs = jnp.stack([src, src + HM]).reshape(2, T, NSUB, NGROUP, NBUF, CH)
    dsts = edge_index[:, 1, :].reshape(T, NSUB, NGROUP, NBUF, CH)

    wl2s = W_l2.reshape(2, 128, O)
    wr2s = W_r2.reshape(2, 128, O)
    bl1 = b_l1.reshape(1, O)
    bl2 = b_l2.reshape(1, O)

    pooled = []
    for h in (0, 1):
        a1, b1 = _mm1(xf, W_l1, W_r1, bl1, h)
        s1, deg = _sc_segsum(a1.reshape(2 * HM, 128), srcs, dsts,
                             h * TSTEPS, with_deg=True)
        a2, b2 = _mm2(s1.reshape(2, HM, 128), b1, deg.reshape(HM, 1),
                      wl2s, wr2s, bl2)
        s2 = _sc_segsum(a2.reshape(2 * HM, 128), srcs, dsts,
                        h * TSTEPS, with_deg=False)
        pooled.append(_pool(s2, b2.reshape(2, TSTEPS, N, 128),
                            deg.reshape(TSTEPS, N, 1)))
    z = jnp.concatenate([p.reshape(TSTEPS, O) for p in pooled], axis=0)

    gru_out, nce, acc = _head(z, W_ih.T, W_hh.T,
                              b_ih.reshape(1, 3 * O), b_hh.reshape(1, 3 * O))
    return nce[0, 0], acc[0, 0], gru_out[None]
